# Initial kernel scaffold; baseline (speedup 1.0000x reference)
import jax
import jax.numpy as jnp
from jax.experimental import pallas as pl


def kernel(x, edge_index, batch, W1, b1, W2, b2, Wt1, bt1, Wt2, bt2, Wg1, bg1, Wg2, bg2):
    raise NotImplementedError("write your pallas kernel here")


# speedup vs baseline: 7.3138x; 7.3138x over previous
import functools

import jax
import jax.numpy as jnp
from jax import lax
from jax.experimental import pallas as pl
from jax.experimental.pallas import tpu as pltpu
from jax.experimental.pallas import tpu_sc as plsc

N = 10000
D = 128
B = 64
T = 8
G = 4

NP = 10240
NC = 2
NS = 16
NTILES = NC * NS
K = 128
NCH = 80
EP = NTILES * NCH * K
RPT = NP // NS

RBLK = 1024
GR = NP // RBLK


def _sc_mesh():
    return plsc.VectorSubcoreMesh(
        core_axis_name="c", subcore_axis_name="s",
        num_cores=NC, num_subcores=NS)


def _sc_deg(dst3, ones_rows, zrows):
    @functools.partial(
        pl.kernel,
        out_type=jax.ShapeDtypeStruct((NC, NP, D), jnp.float32),
        mesh=_sc_mesh(),
        scratch_types=[
            pltpu.VMEM((NCH, K), jnp.int32),
            pltpu.VMEM((K, D), jnp.float32),
            pltpu.VMEM((K, D), jnp.float32),
            pltpu.VMEM_SHARED((NP, D), jnp.float32),
        ],
    )
    def body(dst_hbm, ones_hbm, zeros_hbm, out_hbm, idx_v, ones_v, stage_v, acc):
        c = lax.axis_index("c")
        s = lax.axis_index("s")
        wid = c * NS + s
        pltpu.sync_copy(dst_hbm.at[wid], idx_v)
        pltpu.sync_copy(ones_hbm, ones_v)
        pltpu.sync_copy(zeros_hbm, stage_v)
        for t in range(RPT // K):
            pltpu.sync_copy(stage_v, acc.at[pl.ds(s * RPT + t * K, K)])
        plsc.subcore_barrier()

        def step(j, carry):
            pltpu.sync_copy(ones_v, acc.at[idx_v.at[j]], add=True)
            return carry
        lax.fori_loop(0, NCH, step, 0)
        plsc.subcore_barrier()
        for t in range(RPT // K):
            sl = pl.ds(s * RPT + t * K, K)
            pltpu.sync_copy(acc.at[sl], stage_v)
            pltpu.sync_copy(stage_v, out_hbm.at[c, sl])

    return body(dst3, ones_rows, zrows)


def _sc_agg(hp, src3, dst3, zrows):
    @functools.partial(
        pl.kernel,
        out_type=jax.ShapeDtypeStruct((NC, NP, D), jnp.float32),
        mesh=_sc_mesh(),
        scratch_types=[
            pltpu.VMEM((NCH, K), jnp.int32),
            pltpu.VMEM((NCH, K), jnp.int32),
            pltpu.VMEM((K, D), jnp.float32),
            pltpu.VMEM_SHARED((NP, D), jnp.float32),
        ],
    )
    def body(hp_hbm, src_hbm, dst_hbm, z_hbm, out_hbm,
             isrc, idst, rows0, acc):
        c = lax.axis_index("c")
        s = lax.axis_index("s")
        wid = c * NS + s
        pltpu.sync_copy(src_hbm.at[wid], isrc)
        pltpu.sync_copy(dst_hbm.at[wid], idst)
        pltpu.sync_copy(z_hbm, rows0)
        for t in range(RPT // K):
            pltpu.sync_copy(rows0, acc.at[pl.ds(s * RPT + t * K, K)])
        plsc.subcore_barrier()

        def step(j, carry):
            pltpu.sync_copy(hp_hbm.at[isrc.at[j]], rows0)
            pltpu.sync_copy(rows0, acc.at[idst.at[j]], add=True)
            return carry
        lax.fori_loop(0, NCH, step, 0)

        plsc.subcore_barrier()
        for t in range(RPT // K):
            sl = pl.ds(s * RPT + t * K, K)
            pltpu.sync_copy(acc.at[sl], rows0)
            pltpu.sync_copy(rows0, out_hbm.at[c, sl])

    return body(hp, src3, dst3, zrows)


def _dinv(d0, d1):
    return lax.rsqrt(d0[:, 0:1] + d1[:, 0:1] + 1.0)


def _mm1_body(x_ref, w_ref, d0_ref, d1_ref, o_ref):
    dv = _dinv(d0_ref[...], d1_ref[...])
    o_ref[...] = dv * jnp.dot(x_ref[...], w_ref[...],
                              preferred_element_type=jnp.float32)


def _mm1(xp, W1, d0, d1):
    return pl.pallas_call(
        _mm1_body,
        grid=(GR,),
        in_specs=[
            pl.BlockSpec((RBLK, D), lambda i: (i, 0)),
            pl.BlockSpec((D, D), lambda i: (0, 0)),
            pl.BlockSpec((RBLK, D), lambda i: (i, 0)),
            pl.BlockSpec((RBLK, D), lambda i: (i, 0)),
        ],
        out_specs=pl.BlockSpec((RBLK, D), lambda i: (i, 0)),
        out_shape=jax.ShapeDtypeStruct((NP, D), jnp.float32),
    )(xp, W1, d0, d1)


def _mid_body(a0_ref, a1_ref, hp_ref, d0_ref, d1_ref, b_ref, w_ref, o_ref):
    dv = _dinv(d0_ref[...], d1_ref[...])
    z = jnp.maximum(dv * (a0_ref[...] + a1_ref[...] + hp_ref[...]) + b_ref[...],
                    0.0)
    o_ref[...] = dv * jnp.dot(z, w_ref[...], preferred_element_type=jnp.float32)


def _mid(a0, a1, hp1, d0, d1, b1, W2):
    return pl.pallas_call(
        _mid_body,
        grid=(GR,),
        in_specs=[
            pl.BlockSpec((RBLK, D), lambda i: (i, 0)),
            pl.BlockSpec((RBLK, D), lambda i: (i, 0)),
            pl.BlockSpec((RBLK, D), lambda i: (i, 0)),
            pl.BlockSpec((RBLK, D), lambda i: (i, 0)),
            pl.BlockSpec((RBLK, D), lambda i: (i, 0)),
            pl.BlockSpec((1, D), lambda i: (0, 0)),
            pl.BlockSpec((D, D), lambda i: (0, 0)),
        ],
        out_specs=pl.BlockSpec((RBLK, D), lambda i: (i, 0)),
        out_shape=jax.ShapeDtypeStruct((NP, D), jnp.float32),
    )(a0, a1, hp1, d0, d1, b1, W2)


def _fin_body(a0_ref, a1_ref, hp_ref, d0_ref, d1_ref, b2_ref, bat_ref,
              wt1_ref, bt1_ref, wt2_ref, bt2_ref,
              wg1_ref, bg1_ref, wg2_ref, bg2_ref,
              type_ref, grade_ref, ssum, cnt):
    i = pl.program_id(0)

    @pl.when(i == 0)
    def _init():
        ssum[...] = jnp.zeros_like(ssum)
        cnt[...] = jnp.zeros_like(cnt)

    dv = _dinv(d0_ref[...], d1_ref[...])
    z = jnp.maximum(
        dv * (a0_ref[...] + a1_ref[...] + hp_ref[...]) + b2_ref[...], 0.0)
    bvec = bat_ref[0, 0, :]
    P = (bvec[None, :] == lax.broadcasted_iota(jnp.int32, (B, RBLK), 0)
         ).astype(jnp.float32)
    ssum[...] += jnp.dot(P, z, preferred_element_type=jnp.float32)
    cnt[...] += jnp.broadcast_to(jnp.sum(P, axis=1, keepdims=True), (B, D))

    @pl.when(i == GR - 1)
    def _heads():
        g = ssum[...] / jnp.maximum(cnt[...], 1.0)
        th = jnp.maximum(
            jnp.dot(g, wt1_ref[...], preferred_element_type=jnp.float32)
            + bt1_ref[...], 0.0)
        type_ref[...] = jnp.dot(th, wt2_ref[...],
                                preferred_element_type=jnp.float32) + bt2_ref[...]
        for t in range(T):
            hg = jnp.maximum(
                jnp.dot(g, wg1_ref[t], preferred_element_type=jnp.float32)
                + bg1_ref[t:t + 1, :], 0.0)
            grade_ref[t] = jnp.dot(hg, wg2_ref[t],
                                   preferred_element_type=jnp.float32
                                   ) + bg2_ref[t:t + 1, :]


def _fin(a0, a1, hp2, d0, d1, b2, bat3, Wt1, bt1, Wt2p, bt2p, Wg1, bg1,
         Wg2p, bg2p):
    return pl.pallas_call(
        _fin_body,
        grid=(GR,),
        in_specs=[
            pl.BlockSpec((RBLK, D), lambda i: (i, 0)),
            pl.BlockSpec((RBLK, D), lambda i: (i, 0)),
            pl.BlockSpec((RBLK, D), lambda i: (i, 0)),
            pl.BlockSpec((RBLK, D), lambda i: (i, 0)),
            pl.BlockSpec((RBLK, D), lambda i: (i, 0)),
            pl.BlockSpec((1, D), lambda i: (0, 0)),
            pl.BlockSpec((1, 1, RBLK), lambda i: (i, 0, 0)),
            pl.BlockSpec((D, D), lambda i: (0, 0)),
            pl.BlockSpec((1, D), lambda i: (0, 0)),
            pl.BlockSpec((D, D), lambda i: (0, 0)),
            pl.BlockSpec((1, D), lambda i: (0, 0)),
            pl.BlockSpec((T, D, D), lambda i: (0, 0, 0)),
            pl.BlockSpec((T, D), lambda i: (0, 0)),
            pl.BlockSpec((T, D, D), lambda i: (0, 0, 0)),
            pl.BlockSpec((T, D), lambda i: (0, 0)),
        ],
        out_specs=[
            pl.BlockSpec((B, D), lambda i: (0, 0)),
            pl.BlockSpec((T, B, D), lambda i: (0, 0, 0)),
        ],
        out_shape=[
            jax.ShapeDtypeStruct((B, D), jnp.float32),
            jax.ShapeDtypeStruct((T, B, D), jnp.float32),
        ],
        scratch_shapes=[
            pltpu.VMEM((B, D), jnp.float32),
            pltpu.VMEM((B, D), jnp.float32),
        ],
        compiler_params=pltpu.CompilerParams(
            dimension_semantics=("arbitrary",)),
    )(a0, a1, hp2, d0, d1, b2, bat3, Wt1, bt1, Wt2p, bt2p, Wg1, bg1, Wg2p,
      bg2p)


def kernel(x, edge_index, batch, W1, b1, W2, b2, Wt1, bt1, Wt2, bt2,
           Wg1, bg1, Wg2, bg2):
    f32 = jnp.float32
    src = edge_index[0].astype(jnp.int32)
    dst = edge_index[1].astype(jnp.int32)
    pad_e = EP - src.shape[0]
    srcp = jnp.concatenate(
        [src, jnp.full((pad_e,), N, jnp.int32)]).reshape(NTILES, NCH, K)
    dstp = jnp.concatenate(
        [dst, jnp.full((pad_e,), N, jnp.int32)]).reshape(NTILES, NCH, K)
    xp = jnp.zeros((NP, D), f32).at[:N].set(x.astype(f32))
    batp = jnp.concatenate(
        [batch.astype(jnp.int32), jnp.full((NP - N,), B, jnp.int32)]
    ).reshape(GR, 1, RBLK)
    ones_rows = jnp.ones((K, D), f32)
    zrows = jnp.zeros((K, D), f32)

    degp = _sc_deg(dstp, ones_rows, zrows)
    d0, d1 = degp[0], degp[1]

    hp1 = _mm1(xp, W1, d0, d1)
    aggp1 = _sc_agg(hp1, srcp, dstp, zrows)
    hp2 = _mid(aggp1[0], aggp1[1], hp1, d0, d1, b1.reshape(1, D), W2)
    aggp2 = _sc_agg(hp2, srcp, dstp, zrows)

    Wt2p = jnp.zeros((D, D), f32).at[:, :T].set(Wt2)
    bt2p = jnp.zeros((1, D), f32).at[0, :T].set(bt2)
    Wg2p = jnp.zeros((T, D, D), f32).at[:, :, :G].set(Wg2)
    bg2p = jnp.zeros((T, D), f32).at[:, :G].set(bg2)

    type_full, grade_full = _fin(
        aggp2[0], aggp2[1], hp2, d0, d1, b2.reshape(1, D), batp,
        Wt1, bt1.reshape(1, D), Wt2p, bt2p, Wg1, bg1, Wg2p, bg2p)

    type_logits = type_full[:, :T]
    grade_logits = jnp.transpose(grade_full[:, :, :G], (1, 0, 2))
    return (type_logits, grade_logits)


# speedup vs baseline: 8.0446x; 1.0999x over previous
import functools

import jax
import jax.numpy as jnp
from jax import lax
from jax.experimental import pallas as pl
from jax.experimental.pallas import tpu as pltpu
from jax.experimental.pallas import tpu_sc as plsc

N = 10000
D = 128
B = 64
T = 8
G = 4

NP = 10240
NC = 2
NS = 16
NTILES = NC * NS
K = 128
NCH = 80
EP = NTILES * NCH * K
RPT = NP // NS

RBLK = 1024
GR = NP // RBLK


def _sc_mesh():
    return plsc.VectorSubcoreMesh(
        core_axis_name="c", subcore_axis_name="s",
        num_cores=NC, num_subcores=NS)


def _sc_deg(dst3, ones_rows, zrows):
    @functools.partial(
        pl.kernel,
        out_type=jax.ShapeDtypeStruct((NC, NP, D), jnp.float32),
        mesh=_sc_mesh(),
        scratch_types=[
            pltpu.VMEM((NCH, K), jnp.int32),
            pltpu.VMEM((K, D), jnp.float32),
            pltpu.VMEM((K, D), jnp.float32),
            pltpu.VMEM_SHARED((NP, D), jnp.float32),
        ],
    )
    def body(dst_hbm, ones_hbm, zeros_hbm, out_hbm, idx_v, ones_v, stage_v, acc):
        c = lax.axis_index("c")
        s = lax.axis_index("s")
        wid = c * NS + s
        pltpu.sync_copy(dst_hbm.at[wid], idx_v)
        pltpu.sync_copy(ones_hbm, ones_v)
        pltpu.sync_copy(zeros_hbm, stage_v)
        for t in range(RPT // K):
            pltpu.sync_copy(stage_v, acc.at[pl.ds(s * RPT + t * K, K)])
        plsc.subcore_barrier()

        def step(j, carry):
            pltpu.sync_copy(ones_v, acc.at[idx_v.at[j]], add=True)
            return carry
        lax.fori_loop(0, NCH, step, 0)
        plsc.subcore_barrier()
        for t in range(RPT // K):
            sl = pl.ds(s * RPT + t * K, K)
            pltpu.sync_copy(acc.at[sl], stage_v)
            pltpu.sync_copy(stage_v, out_hbm.at[c, sl])

    return body(dst3, ones_rows, zrows)


def _sc_agg(hp, src3, dst3, zrows):
    @functools.partial(
        pl.kernel,
        out_type=jax.ShapeDtypeStruct((NC, NP, D), jnp.float32),
        mesh=_sc_mesh(),
        scratch_types=[
            pltpu.VMEM((NCH, K), jnp.int32),
            pltpu.VMEM((K,), jnp.int32),
            pltpu.VMEM((K,), jnp.int32),
            pltpu.VMEM((K, D), jnp.float32),
            pltpu.VMEM((K, D), jnp.float32),
            pltpu.VMEM_SHARED((NP, D), jnp.float32),
            pltpu.SemaphoreType.DMA,
            pltpu.SemaphoreType.DMA,
            pltpu.SemaphoreType.DMA,
            pltpu.SemaphoreType.DMA,
        ],
    )
    def body(hp_hbm, src_hbm, dst_hbm, z_hbm, out_hbm,
             isrc, d0, d1, rows0, rows1, acc, g0, g1, id0, id1):
        c = lax.axis_index("c")
        s = lax.axis_index("s")
        wid = c * NS + s
        pltpu.sync_copy(src_hbm.at[wid], isrc)
        pltpu.sync_copy(z_hbm, rows0)
        for t in range(RPT // K):
            pltpu.sync_copy(rows0, acc.at[pl.ds(s * RPT + t * K, K)])
        plsc.subcore_barrier()

        pltpu.async_copy(dst_hbm.at[wid, 0], d0, id0)
        pltpu.async_copy(dst_hbm.at[wid, 1], d1, id1)
        pltpu.async_copy(hp_hbm.at[isrc.at[0]], rows0, g0)
        pltpu.async_copy(hp_hbm.at[isrc.at[1]], rows1, g1)

        def step(jj, carry):
            j = jj * 2
            pltpu.make_async_copy(dst_hbm.at[wid, j], d0, id0).wait()
            pltpu.make_async_copy(hp_hbm.at[isrc.at[j]], rows0, g0).wait()
            pltpu.sync_copy(rows0, acc.at[d0], add=True)

            @pl.when(jj < NCH // 2 - 1)
            def _next0():
                pltpu.async_copy(dst_hbm.at[wid, j + 2], d0, id0)
                pltpu.async_copy(hp_hbm.at[isrc.at[j + 2]], rows0, g0)

            pltpu.make_async_copy(dst_hbm.at[wid, j + 1], d1, id1).wait()
            pltpu.make_async_copy(hp_hbm.at[isrc.at[j + 1]], rows1, g1).wait()
            pltpu.sync_copy(rows1, acc.at[d1], add=True)

            @pl.when(jj < NCH // 2 - 1)
            def _next1():
                pltpu.async_copy(dst_hbm.at[wid, j + 3], d1, id1)
                pltpu.async_copy(hp_hbm.at[isrc.at[j + 3]], rows1, g1)
            return carry
        lax.fori_loop(0, NCH // 2, step, 0)

        plsc.subcore_barrier()
        for t in range(RPT // K):
            sl = pl.ds(s * RPT + t * K, K)
            pltpu.sync_copy(acc.at[sl], rows0)
            pltpu.sync_copy(rows0, out_hbm.at[c, sl])

    return body(hp, src3, dst3, zrows)


def _dinv(d0, d1):
    return lax.rsqrt(d0[:, 0:1] + d1[:, 0:1] + 1.0)


def _mm1_body(x_ref, w_ref, d0_ref, d1_ref, o_ref):
    dv = _dinv(d0_ref[...], d1_ref[...])
    o_ref[...] = dv * jnp.dot(x_ref[...], w_ref[...],
                              preferred_element_type=jnp.float32)


def _mm1(xp, W1, d0, d1):
    return pl.pallas_call(
        _mm1_body,
        grid=(GR,),
        in_specs=[
            pl.BlockSpec((RBLK, D), lambda i: (i, 0)),
            pl.BlockSpec((D, D), lambda i: (0, 0)),
            pl.BlockSpec((RBLK, D), lambda i: (i, 0)),
            pl.BlockSpec((RBLK, D), lambda i: (i, 0)),
        ],
        out_specs=pl.BlockSpec((RBLK, D), lambda i: (i, 0)),
        out_shape=jax.ShapeDtypeStruct((NP, D), jnp.float32),
    )(xp, W1, d0, d1)


def _mid_body(a0_ref, a1_ref, hp_ref, d0_ref, d1_ref, b_ref, w_ref, o_ref):
    dv = _dinv(d0_ref[...], d1_ref[...])
    z = jnp.maximum(dv * (a0_ref[...] + a1_ref[...] + hp_ref[...]) + b_ref[...],
                    0.0)
    o_ref[...] = dv * jnp.dot(z, w_ref[...], preferred_element_type=jnp.float32)


def _mid(a0, a1, hp1, d0, d1, b1, W2):
    return pl.pallas_call(
        _mid_body,
        grid=(GR,),
        in_specs=[
            pl.BlockSpec((RBLK, D), lambda i: (i, 0)),
            pl.BlockSpec((RBLK, D), lambda i: (i, 0)),
            pl.BlockSpec((RBLK, D), lambda i: (i, 0)),
            pl.BlockSpec((RBLK, D), lambda i: (i, 0)),
            pl.BlockSpec((RBLK, D), lambda i: (i, 0)),
            pl.BlockSpec((1, D), lambda i: (0, 0)),
            pl.BlockSpec((D, D), lambda i: (0, 0)),
        ],
        out_specs=pl.BlockSpec((RBLK, D), lambda i: (i, 0)),
        out_shape=jax.ShapeDtypeStruct((NP, D), jnp.float32),
    )(a0, a1, hp1, d0, d1, b1, W2)


def _fin_body(a0_ref, a1_ref, hp_ref, d0_ref, d1_ref, b2_ref, bat_ref,
              wt1_ref, bt1_ref, wt2_ref, bt2_ref,
              wg1_ref, bg1_ref, wg2_ref, bg2_ref,
              type_ref, grade_ref, ssum, cnt):
    i = pl.program_id(0)

    @pl.when(i == 0)
    def _init():
        ssum[...] = jnp.zeros_like(ssum)
        cnt[...] = jnp.zeros_like(cnt)

    dv = _dinv(d0_ref[...], d1_ref[...])
    z = jnp.maximum(
        dv * (a0_ref[...] + a1_ref[...] + hp_ref[...]) + b2_ref[...], 0.0)
    bvec = bat_ref[0, 0, :]
    P = (bvec[None, :] == lax.broadcasted_iota(jnp.int32, (B, RBLK), 0)
         ).astype(jnp.float32)
    ssum[...] += jnp.dot(P, z, preferred_element_type=jnp.float32)
    cnt[...] += jnp.broadcast_to(jnp.sum(P, axis=1, keepdims=True), (B, D))

    @pl.when(i == GR - 1)
    def _heads():
        g = ssum[...] / jnp.maximum(cnt[...], 1.0)
        th = jnp.maximum(
            jnp.dot(g, wt1_ref[...], preferred_element_type=jnp.float32)
            + bt1_ref[...], 0.0)
        type_ref[...] = jnp.dot(th, wt2_ref[...],
                                preferred_element_type=jnp.float32) + bt2_ref[...]
        for t in range(T):
            hg = jnp.maximum(
                jnp.dot(g, wg1_ref[t], preferred_element_type=jnp.float32)
                + bg1_ref[t:t + 1, :], 0.0)
            grade_ref[t] = jnp.dot(hg, wg2_ref[t],
                                   preferred_element_type=jnp.float32
                                   ) + bg2_ref[t:t + 1, :]


def _fin(a0, a1, hp2, d0, d1, b2, bat3, Wt1, bt1, Wt2p, bt2p, Wg1, bg1,
         Wg2p, bg2p):
    return pl.pallas_call(
        _fin_body,
        grid=(GR,),
        in_specs=[
            pl.BlockSpec((RBLK, D), lambda i: (i, 0)),
            pl.BlockSpec((RBLK, D), lambda i: (i, 0)),
            pl.BlockSpec((RBLK, D), lambda i: (i, 0)),
            pl.BlockSpec((RBLK, D), lambda i: (i, 0)),
            pl.BlockSpec((RBLK, D), lambda i: (i, 0)),
            pl.BlockSpec((1, D), lambda i: (0, 0)),
            pl.BlockSpec((1, 1, RBLK), lambda i: (i, 0, 0)),
            pl.BlockSpec((D, D), lambda i: (0, 0)),
            pl.BlockSpec((1, D), lambda i: (0, 0)),
            pl.BlockSpec((D, D), lambda i: (0, 0)),
            pl.BlockSpec((1, D), lambda i: (0, 0)),
            pl.BlockSpec((T, D, D), lambda i: (0, 0, 0)),
            pl.BlockSpec((T, D), lambda i: (0, 0)),
            pl.BlockSpec((T, D, D), lambda i: (0, 0, 0)),
            pl.BlockSpec((T, D), lambda i: (0, 0)),
        ],
        out_specs=[
            pl.BlockSpec((B, D), lambda i: (0, 0)),
            pl.BlockSpec((T, B, D), lambda i: (0, 0, 0)),
        ],
        out_shape=[
            jax.ShapeDtypeStruct((B, D), jnp.float32),
            jax.ShapeDtypeStruct((T, B, D), jnp.float32),
        ],
        scratch_shapes=[
            pltpu.VMEM((B, D), jnp.float32),
            pltpu.VMEM((B, D), jnp.float32),
        ],
        compiler_params=pltpu.CompilerParams(
            dimension_semantics=("arbitrary",)),
    )(a0, a1, hp2, d0, d1, b2, bat3, Wt1, bt1, Wt2p, bt2p, Wg1, bg1, Wg2p,
      bg2p)


def kernel(x, edge_index, batch, W1, b1, W2, b2, Wt1, bt1, Wt2, bt2,
           Wg1, bg1, Wg2, bg2):
    f32 = jnp.float32
    src = edge_index[0].astype(jnp.int32)
    dst = edge_index[1].astype(jnp.int32)
    pad_e = EP - src.shape[0]
    srcp = jnp.concatenate(
        [src, jnp.full((pad_e,), N, jnp.int32)]).reshape(NTILES, NCH, K)
    dstp = jnp.concatenate(
        [dst, jnp.full((pad_e,), N, jnp.int32)]).reshape(NTILES, NCH, K)
    xp = jnp.zeros((NP, D), f32).at[:N].set(x.astype(f32))
    batp = jnp.concatenate(
        [batch.astype(jnp.int32), jnp.full((NP - N,), B, jnp.int32)]
    ).reshape(GR, 1, RBLK)
    ones_rows = jnp.ones((K, D), f32)
    zrows = jnp.zeros((K, D), f32)

    degp = _sc_deg(dstp, ones_rows, zrows)
    d0, d1 = degp[0], degp[1]

    hp1 = _mm1(xp, W1, d0, d1)
    aggp1 = _sc_agg(hp1, srcp, dstp, zrows)
    hp2 = _mid(aggp1[0], aggp1[1], hp1, d0, d1, b1.reshape(1, D), W2)
    aggp2 = _sc_agg(hp2, srcp, dstp, zrows)

    Wt2p = jnp.zeros((D, D), f32).at[:, :T].set(Wt2)
    bt2p = jnp.zeros((1, D), f32).at[0, :T].set(bt2)
    Wg2p = jnp.zeros((T, D, D), f32).at[:, :, :G].set(Wg2)
    bg2p = jnp.zeros((T, D), f32).at[:, :G].set(bg2)

    type_full, grade_full = _fin(
        aggp2[0], aggp2[1], hp2, d0, d1, b2.reshape(1, D), batp,
        Wt1, bt1.reshape(1, D), Wt2p, bt2p, Wg1, bg1, Wg2p, bg2p)

    type_logits = type_full[:, :T]
    grade_logits = jnp.transpose(grade_full[:, :, :G], (1, 0, 2))
    return (type_logits, grade_logits)


# speedup vs baseline: 25.9428x; 3.2249x over previous
import functools

import jax
import jax.numpy as jnp
from jax import lax
from jax.experimental import pallas as pl
from jax.experimental.pallas import tpu as pltpu
from jax.experimental.pallas import tpu_sc as plsc

N = 10000
D = 128
B = 64
T = 8
G = 4

NP = 10240
NC = 2
NS = 16
NTILES = NC * NS
K = 128
NCH = 80
EP = NTILES * NCH * K
RPT = NP // NS

RBLK = 1024
GR = NP // RBLK


def _sc_mesh():
    return plsc.VectorSubcoreMesh(
        core_axis_name="c", subcore_axis_name="s",
        num_cores=NC, num_subcores=NS)


def _sc_deg(dst3, ones_rows, zrows):
    @functools.partial(
        pl.kernel,
        out_type=jax.ShapeDtypeStruct((NC, NP, D), jnp.float32),
        mesh=_sc_mesh(),
        scratch_types=[
            pltpu.VMEM((NCH, K), jnp.int32),
            pltpu.VMEM((K, D), jnp.float32),
            pltpu.VMEM((K, D), jnp.float32),
            pltpu.VMEM_SHARED((NP, D), jnp.float32),
        ],
    )
    def body(dst_hbm, ones_hbm, zeros_hbm, out_hbm, idx_v, ones_v, stage_v, acc):
        c = lax.axis_index("c")
        s = lax.axis_index("s")
        wid = c * NS + s
        pltpu.sync_copy(dst_hbm.at[wid], idx_v)
        pltpu.sync_copy(ones_hbm, ones_v)
        pltpu.sync_copy(zeros_hbm, stage_v)
        for t in range(RPT // K):
            pltpu.sync_copy(stage_v, acc.at[pl.ds(s * RPT + t * K, K)])
        plsc.subcore_barrier()

        def step(j, carry):
            pltpu.sync_copy(ones_v, acc.at[idx_v.at[j]], add=True)
            return carry
        lax.fori_loop(0, NCH, step, 0)
        plsc.subcore_barrier()
        for t in range(RPT // K):
            sl = pl.ds(s * RPT + t * K, K)
            pltpu.sync_copy(acc.at[sl], stage_v)
            pltpu.sync_copy(stage_v, out_hbm.at[c, sl])

    return body(dst3, ones_rows, zrows)


def _sc_agg(hp, src3, dst3, zrows):
    @functools.partial(
        pl.kernel,
        out_type=jax.ShapeDtypeStruct((NC, NP, D), jnp.float32),
        mesh=_sc_mesh(),
        scratch_types=[
            pltpu.VMEM((NCH, K), jnp.int32),
            pltpu.VMEM((K,), jnp.int32),
            pltpu.VMEM((K,), jnp.int32),
            pltpu.VMEM((K, D), jnp.float32),
            pltpu.VMEM((K, D), jnp.float32),
            pltpu.VMEM_SHARED((NP, D), jnp.float32),
            pltpu.SemaphoreType.DMA,
            pltpu.SemaphoreType.DMA,
            pltpu.SemaphoreType.DMA,
            pltpu.SemaphoreType.DMA,
        ],
    )
    def body(hp_hbm, src_hbm, dst_hbm, z_hbm, out_hbm,
             isrc, d0, d1, rows0, rows1, acc, g0, g1, id0, id1):
        c = lax.axis_index("c")
        s = lax.axis_index("s")
        wid = c * NS + s
        pltpu.sync_copy(src_hbm.at[wid], isrc)
        pltpu.sync_copy(z_hbm, rows0)
        for t in range(RPT // K):
            pltpu.sync_copy(rows0, acc.at[pl.ds(s * RPT + t * K, K)])
        plsc.subcore_barrier()

        pltpu.async_copy(dst_hbm.at[wid, 0], d0, id0)
        pltpu.async_copy(dst_hbm.at[wid, 1], d1, id1)
        pltpu.async_copy(hp_hbm.at[isrc.at[0]], rows0, g0)
        pltpu.async_copy(hp_hbm.at[isrc.at[1]], rows1, g1)

        def step(jj, carry):
            j = jj * 2
            pltpu.make_async_copy(dst_hbm.at[wid, j], d0, id0).wait()
            pltpu.make_async_copy(hp_hbm.at[isrc.at[j]], rows0, g0).wait()
            pltpu.sync_copy(rows0, acc.at[d0], add=True)

            @pl.when(jj < NCH // 2 - 1)
            def _next0():
                pltpu.async_copy(dst_hbm.at[wid, j + 2], d0, id0)
                pltpu.async_copy(hp_hbm.at[isrc.at[j + 2]], rows0, g0)

            pltpu.make_async_copy(dst_hbm.at[wid, j + 1], d1, id1).wait()
            pltpu.make_async_copy(hp_hbm.at[isrc.at[j + 1]], rows1, g1).wait()
            pltpu.sync_copy(rows1, acc.at[d1], add=True)

            @pl.when(jj < NCH // 2 - 1)
            def _next1():
                pltpu.async_copy(dst_hbm.at[wid, j + 3], d1, id1)
                pltpu.async_copy(hp_hbm.at[isrc.at[j + 3]], rows1, g1)
            return carry
        lax.fori_loop(0, NCH // 2, step, 0)

        plsc.subcore_barrier()
        for t in range(RPT // K):
            sl = pl.ds(s * RPT + t * K, K)
            pltpu.sync_copy(acc.at[sl], rows0)
            pltpu.sync_copy(rows0, out_hbm.at[c, sl])

    return body(hp, src3, dst3, zrows)


def _dinv(d0, d1):
    return lax.rsqrt(d0[:, 0:1] + d1[:, 0:1] + 1.0)


def _mm1_body(x_ref, w_ref, d0_ref, d1_ref, o_ref):
    dv = _dinv(d0_ref[...], d1_ref[...])
    o_ref[...] = dv * jnp.dot(x_ref[...], w_ref[...],
                              preferred_element_type=jnp.float32)


def _mm1(xp, W1, d0, d1):
    return pl.pallas_call(
        _mm1_body,
        grid=(GR,),
        in_specs=[
            pl.BlockSpec((RBLK, D), lambda i: (i, 0)),
            pl.BlockSpec((D, D), lambda i: (0, 0)),
            pl.BlockSpec((RBLK, D), lambda i: (i, 0)),
            pl.BlockSpec((RBLK, D), lambda i: (i, 0)),
        ],
        out_specs=pl.BlockSpec((RBLK, D), lambda i: (i, 0)),
        out_shape=jax.ShapeDtypeStruct((NP, D), jnp.float32),
    )(xp, W1, d0, d1)


def _mid_body(a0_ref, a1_ref, hp_ref, d0_ref, d1_ref, b_ref, w_ref, o_ref):
    dv = _dinv(d0_ref[...], d1_ref[...])
    z = jnp.maximum(dv * (a0_ref[...] + a1_ref[...] + hp_ref[...]) + b_ref[...],
                    0.0)
    o_ref[...] = dv * jnp.dot(z, w_ref[...], preferred_element_type=jnp.float32)


def _mid(a0, a1, hp1, d0, d1, b1, W2):
    return pl.pallas_call(
        _mid_body,
        grid=(GR,),
        in_specs=[
            pl.BlockSpec((RBLK, D), lambda i: (i, 0)),
            pl.BlockSpec((RBLK, D), lambda i: (i, 0)),
            pl.BlockSpec((RBLK, D), lambda i: (i, 0)),
            pl.BlockSpec((RBLK, D), lambda i: (i, 0)),
            pl.BlockSpec((RBLK, D), lambda i: (i, 0)),
            pl.BlockSpec((1, D), lambda i: (0, 0)),
            pl.BlockSpec((D, D), lambda i: (0, 0)),
        ],
        out_specs=pl.BlockSpec((RBLK, D), lambda i: (i, 0)),
        out_shape=jax.ShapeDtypeStruct((NP, D), jnp.float32),
    )(a0, a1, hp1, d0, d1, b1, W2)


def _fin_body(a0_ref, a1_ref, hp_ref, d0_ref, d1_ref, b2_ref, bat_ref,
              wt1_ref, bt1_ref, wt2_ref, bt2_ref,
              wg1_ref, bg1_ref, wg2_ref, bg2_ref,
              type_ref, grade_ref, ssum, cnt):
    i = pl.program_id(0)

    @pl.when(i == 0)
    def _init():
        ssum[...] = jnp.zeros_like(ssum)
        cnt[...] = jnp.zeros_like(cnt)

    dv = _dinv(d0_ref[...], d1_ref[...])
    z = jnp.maximum(
        dv * (a0_ref[...] + a1_ref[...] + hp_ref[...]) + b2_ref[...], 0.0)
    bvec = bat_ref[0, 0, :]
    P = (bvec[None, :] == lax.broadcasted_iota(jnp.int32, (B, RBLK), 0)
         ).astype(jnp.float32)
    ssum[...] += jnp.dot(P, z, preferred_element_type=jnp.float32)
    cnt[...] += jnp.broadcast_to(jnp.sum(P, axis=1, keepdims=True), (B, D))

    @pl.when(i == GR - 1)
    def _heads():
        g = ssum[...] / jnp.maximum(cnt[...], 1.0)
        th = jnp.maximum(
            jnp.dot(g, wt1_ref[...], preferred_element_type=jnp.float32)
            + bt1_ref[...], 0.0)
        type_ref[...] = jnp.dot(th, wt2_ref[...],
                                preferred_element_type=jnp.float32) + bt2_ref[...]
        for t in range(T):
            hg = jnp.maximum(
                jnp.dot(g, wg1_ref[t], preferred_element_type=jnp.float32)
                + bg1_ref[t:t + 1, :], 0.0)
            grade_ref[t] = jnp.dot(hg, wg2_ref[t],
                                   preferred_element_type=jnp.float32
                                   ) + bg2_ref[t:t + 1, :]


def _fin(a0, a1, hp2, d0, d1, b2, bat3, Wt1, bt1, Wt2p, bt2p, Wg1, bg1,
         Wg2p, bg2p):
    return pl.pallas_call(
        _fin_body,
        grid=(GR,),
        in_specs=[
            pl.BlockSpec((RBLK, D), lambda i: (i, 0)),
            pl.BlockSpec((RBLK, D), lambda i: (i, 0)),
            pl.BlockSpec((RBLK, D), lambda i: (i, 0)),
            pl.BlockSpec((RBLK, D), lambda i: (i, 0)),
            pl.BlockSpec((RBLK, D), lambda i: (i, 0)),
            pl.BlockSpec((1, D), lambda i: (0, 0)),
            pl.BlockSpec((1, 1, RBLK), lambda i: (i, 0, 0)),
            pl.BlockSpec((D, D), lambda i: (0, 0)),
            pl.BlockSpec((1, D), lambda i: (0, 0)),
            pl.BlockSpec((D, D), lambda i: (0, 0)),
            pl.BlockSpec((1, D), lambda i: (0, 0)),
            pl.BlockSpec((T, D, D), lambda i: (0, 0, 0)),
            pl.BlockSpec((T, D), lambda i: (0, 0)),
            pl.BlockSpec((T, D, D), lambda i: (0, 0, 0)),
            pl.BlockSpec((T, D), lambda i: (0, 0)),
        ],
        out_specs=[
            pl.BlockSpec((B, D), lambda i: (0, 0)),
            pl.BlockSpec((T, B, D), lambda i: (0, 0, 0)),
        ],
        out_shape=[
            jax.ShapeDtypeStruct((B, D), jnp.float32),
            jax.ShapeDtypeStruct((T, B, D), jnp.float32),
        ],
        scratch_shapes=[
            pltpu.VMEM((B, D), jnp.float32),
            pltpu.VMEM((B, D), jnp.float32),
        ],
        compiler_params=pltpu.CompilerParams(
            dimension_semantics=("arbitrary",)),
    )(a0, a1, hp2, d0, d1, b2, bat3, Wt1, bt1, Wt2p, bt2p, Wg1, bg1, Wg2p,
      bg2p)


def kernel(x, edge_index, batch, W1, b1, W2, b2, Wt1, bt1, Wt2, bt2,
           Wg1, bg1, Wg2, bg2):
    f32 = jnp.float32
    src = edge_index[0].astype(jnp.int32)
    dst = edge_index[1].astype(jnp.int32)
    pad_e = EP - src.shape[0]
    pad_idx = N + jnp.arange(pad_e, dtype=jnp.int32) % (NP - N)
    srcp = jnp.concatenate([src, pad_idx]).reshape(NTILES, NCH, K)
    dstp = jnp.concatenate([dst, pad_idx]).reshape(NTILES, NCH, K)
    xp = jnp.zeros((NP, D), f32).at[:N].set(x.astype(f32))
    batp = jnp.concatenate(
        [batch.astype(jnp.int32), jnp.full((NP - N,), B, jnp.int32)]
    ).reshape(GR, 1, RBLK)
    ones_rows = jnp.ones((K, D), f32)
    zrows = jnp.zeros((K, D), f32)

    degp = _sc_deg(dstp, ones_rows, zrows)
    d0, d1 = degp[0], degp[1]

    hp1 = _mm1(xp, W1, d0, d1)
    aggp1 = _sc_agg(hp1, srcp, dstp, zrows)
    hp2 = _mid(aggp1[0], aggp1[1], hp1, d0, d1, b1.reshape(1, D), W2)
    aggp2 = _sc_agg(hp2, srcp, dstp, zrows)

    Wt2p = jnp.zeros((D, D), f32).at[:, :T].set(Wt2)
    bt2p = jnp.zeros((1, D), f32).at[0, :T].set(bt2)
    Wg2p = jnp.zeros((T, D, D), f32).at[:, :, :G].set(Wg2)
    bg2p = jnp.zeros((T, D), f32).at[:, :G].set(bg2)

    type_full, grade_full = _fin(
        aggp2[0], aggp2[1], hp2, d0, d1, b2.reshape(1, D), batp,
        Wt1, bt1.reshape(1, D), Wt2p, bt2p, Wg1, bg1, Wg2p, bg2p)

    type_logits = type_full[:, :T]
    grade_logits = jnp.transpose(grade_full[:, :, :G], (1, 0, 2))
    return (type_logits, grade_logits)


# speedup vs baseline: 26.1790x; 1.0091x over previous
import functools

import jax
import jax.numpy as jnp
from jax import lax
from jax.experimental import pallas as pl
from jax.experimental.pallas import tpu as pltpu
from jax.experimental.pallas import tpu_sc as plsc

N = 10000
E = 320000
D = 128
B = 64
T = 8
G = 4

NP = 10240
NC = 2
NS = 16
NTILES = NC * NS
K = 128
EPT = 10240
NCH = EPT // K
NCH_LAST = (E - (NTILES - 1) * EPT) // K
RPT = NP // NS

RBLK = 1024
GR = NP // RBLK


def _sc_mesh():
    return plsc.VectorSubcoreMesh(
        core_axis_name="c", subcore_axis_name="s",
        num_cores=NC, num_subcores=NS)


def _sc_deg(ei, ones_rows, zrows):
    @functools.partial(
        pl.kernel,
        out_type=jax.ShapeDtypeStruct((NC, NP, D), jnp.float32),
        mesh=_sc_mesh(),
        scratch_types=[
            pltpu.VMEM((K,), jnp.int32),
            pltpu.VMEM((K,), jnp.int32),
            pltpu.VMEM((K, D), jnp.float32),
            pltpu.VMEM((K, D), jnp.float32),
            pltpu.VMEM_SHARED((NP, D), jnp.float32),
            pltpu.SemaphoreType.DMA,
            pltpu.SemaphoreType.DMA,
        ],
    )
    def body(ei_hbm, ones_hbm, zeros_hbm, out_hbm,
             d0, d1, ones_v, stage_v, acc, id0, id1):
        c = lax.axis_index("c")
        s = lax.axis_index("s")
        wid = c * NS + s
        base = wid * EPT
        npair = jnp.where(wid == NTILES - 1, NCH_LAST // 2, NCH // 2)
        pltpu.sync_copy(ones_hbm, ones_v)
        pltpu.sync_copy(zeros_hbm, stage_v)
        for t in range(RPT // K):
            pltpu.sync_copy(stage_v, acc.at[pl.ds(s * RPT + t * K, K)])
        plsc.subcore_barrier()

        pltpu.async_copy(ei_hbm.at[1, pl.ds(base, K)], d0, id0)
        pltpu.async_copy(ei_hbm.at[1, pl.ds(base + K, K)], d1, id1)

        def step(jj, carry):
            j = jj * 2
            pltpu.make_async_copy(ei_hbm.at[1, pl.ds(base, K)], d0, id0).wait()
            pltpu.sync_copy(ones_v, acc.at[d0], add=True)

            @pl.when(jj < npair - 1)
            def _next0():
                pltpu.async_copy(
                    ei_hbm.at[1, pl.ds(base + (j + 2) * K, K)], d0, id0)

            pltpu.make_async_copy(ei_hbm.at[1, pl.ds(base, K)], d1, id1).wait()
            pltpu.sync_copy(ones_v, acc.at[d1], add=True)

            @pl.when(jj < npair - 1)
            def _next1():
                pltpu.async_copy(
                    ei_hbm.at[1, pl.ds(base + (j + 3) * K, K)], d1, id1)
            return carry
        lax.fori_loop(0, npair, step, 0)
        plsc.subcore_barrier()
        for t in range(RPT // K):
            sl = pl.ds(s * RPT + t * K, K)
            pltpu.sync_copy(acc.at[sl], stage_v)
            pltpu.sync_copy(stage_v, out_hbm.at[c, sl])

    return body(ei, ones_rows, zrows)


def _sc_agg(hp, ei, zrows):
    @functools.partial(
        pl.kernel,
        out_type=jax.ShapeDtypeStruct((NC, NP, D), jnp.float32),
        mesh=_sc_mesh(),
        scratch_types=[
            pltpu.VMEM((K,), jnp.int32),
            pltpu.VMEM((K,), jnp.int32),
            pltpu.VMEM((K,), jnp.int32),
            pltpu.VMEM((K,), jnp.int32),
            pltpu.VMEM((K, D), jnp.float32),
            pltpu.VMEM((K, D), jnp.float32),
            pltpu.VMEM_SHARED((NP, D), jnp.float32),
            pltpu.SemaphoreType.DMA,
            pltpu.SemaphoreType.DMA,
            pltpu.SemaphoreType.DMA,
            pltpu.SemaphoreType.DMA,
            pltpu.SemaphoreType.DMA,
            pltpu.SemaphoreType.DMA,
        ],
    )
    def body(hp_hbm, ei_hbm, z_hbm, out_hbm,
             s0, s1, d0, d1, rows0, rows1, acc, g0, g1, is0, is1, id0, id1):
        c = lax.axis_index("c")
        s = lax.axis_index("s")
        wid = c * NS + s
        base = wid * EPT
        npair = jnp.where(wid == NTILES - 1, NCH_LAST // 2, NCH // 2)
        pltpu.sync_copy(z_hbm, rows0)
        for t in range(RPT // K):
            pltpu.sync_copy(rows0, acc.at[pl.ds(s * RPT + t * K, K)])
        plsc.subcore_barrier()

        pltpu.async_copy(ei_hbm.at[0, pl.ds(base, K)], s0, is0)
        pltpu.async_copy(ei_hbm.at[0, pl.ds(base + K, K)], s1, is1)
        pltpu.async_copy(ei_hbm.at[1, pl.ds(base, K)], d0, id0)
        pltpu.async_copy(ei_hbm.at[1, pl.ds(base + K, K)], d1, id1)
        pltpu.make_async_copy(ei_hbm.at[0, pl.ds(base, K)], s0, is0).wait()
        pltpu.async_copy(hp_hbm.at[s0], rows0, g0)
        pltpu.make_async_copy(ei_hbm.at[0, pl.ds(base, K)], s1, is1).wait()
        pltpu.async_copy(hp_hbm.at[s1], rows1, g1)

        def step(jj, carry):
            j = jj * 2
            more = jj < npair - 1
            pltpu.make_async_copy(ei_hbm.at[1, pl.ds(base, K)], d0, id0).wait()
            pltpu.make_async_copy(hp_hbm.at[s0], rows0, g0).wait()
            pltpu.sync_copy(rows0, acc.at[d0], add=True)

            @pl.when(more)
            def _pref0():
                pltpu.async_copy(
                    ei_hbm.at[0, pl.ds(base + (j + 2) * K, K)], s0, is0)
                pltpu.async_copy(
                    ei_hbm.at[1, pl.ds(base + (j + 2) * K, K)], d0, id0)

            pltpu.make_async_copy(ei_hbm.at[1, pl.ds(base, K)], d1, id1).wait()
            pltpu.make_async_copy(hp_hbm.at[s1], rows1, g1).wait()

            @pl.when(more)
            def _go0():
                pltpu.make_async_copy(
                    ei_hbm.at[0, pl.ds(base, K)], s0, is0).wait()
                pltpu.async_copy(hp_hbm.at[s0], rows0, g0)

            pltpu.sync_copy(rows1, acc.at[d1], add=True)

            @pl.when(more)
            def _go1():
                pltpu.async_copy(
                    ei_hbm.at[0, pl.ds(base + (j + 3) * K, K)], s1, is1)
                pltpu.async_copy(
                    ei_hbm.at[1, pl.ds(base + (j + 3) * K, K)], d1, id1)
                pltpu.make_async_copy(
                    ei_hbm.at[0, pl.ds(base, K)], s1, is1).wait()
                pltpu.async_copy(hp_hbm.at[s1], rows1, g1)
            return carry
        lax.fori_loop(0, npair, step, 0)

        plsc.subcore_barrier()
        for t in range(RPT // K):
            sl = pl.ds(s * RPT + t * K, K)
            pltpu.sync_copy(acc.at[sl], rows0)
            pltpu.sync_copy(rows0, out_hbm.at[c, sl])

    return body(hp, ei, zrows)


def _mm1_body(x_ref, w_ref, dp_ref, o_ref, dv_ref):
    deg = dp_ref[0, :, 0:1] + dp_ref[1, :, 0:1] + 1.0
    dv = lax.rsqrt(deg)
    dv_ref[...] = dv
    o_ref[...] = dv * jnp.dot(x_ref[...], w_ref[...],
                              preferred_element_type=jnp.float32)


def _mm1(xp, W1, degp):
    return pl.pallas_call(
        _mm1_body,
        grid=(GR,),
        in_specs=[
            pl.BlockSpec((RBLK, D), lambda i: (i, 0)),
            pl.BlockSpec((D, D), lambda i: (0, 0)),
            pl.BlockSpec((NC, RBLK, D), lambda i: (0, i, 0)),
        ],
        out_specs=[
            pl.BlockSpec((RBLK, D), lambda i: (i, 0)),
            pl.BlockSpec((RBLK, 1), lambda i: (i, 0)),
        ],
        out_shape=[
            jax.ShapeDtypeStruct((NP, D), jnp.float32),
            jax.ShapeDtypeStruct((NP, 1), jnp.float32),
        ],
    )(xp, W1, degp)


def _mid_body(ap_ref, hp_ref, dv_ref, b_ref, w_ref, o_ref):
    dv = dv_ref[...]
    z = jnp.maximum(
        dv * (ap_ref[0] + ap_ref[1] + hp_ref[...]) + b_ref[...], 0.0)
    o_ref[...] = dv * jnp.dot(z, w_ref[...], preferred_element_type=jnp.float32)


def _mid(aggp, hp1, dv, b1, W2):
    return pl.pallas_call(
        _mid_body,
        grid=(GR,),
        in_specs=[
            pl.BlockSpec((NC, RBLK, D), lambda i: (0, i, 0)),
            pl.BlockSpec((RBLK, D), lambda i: (i, 0)),
            pl.BlockSpec((RBLK, 1), lambda i: (i, 0)),
            pl.BlockSpec((1, D), lambda i: (0, 0)),
            pl.BlockSpec((D, D), lambda i: (0, 0)),
        ],
        out_specs=pl.BlockSpec((RBLK, D), lambda i: (i, 0)),
        out_shape=jax.ShapeDtypeStruct((NP, D), jnp.float32),
    )(aggp, hp1, dv, b1, W2)


def _fin_body(ap_ref, hp_ref, dv_ref, b2_ref, bat_ref,
              wt1_ref, bt1_ref, wt2_ref, bt2_ref,
              wg1_ref, bg1_ref, wg2_ref, bg2_ref,
              type_ref, grade_ref, ssum, cnt):
    i = pl.program_id(0)

    @pl.when(i == 0)
    def _init():
        ssum[...] = jnp.zeros_like(ssum)
        cnt[...] = jnp.zeros_like(cnt)

    dv = dv_ref[...]
    z = jnp.maximum(
        dv * (ap_ref[0] + ap_ref[1] + hp_ref[...]) + b2_ref[...], 0.0)
    bvec = bat_ref[0, 0, :]
    P = (bvec[None, :] == lax.broadcasted_iota(jnp.int32, (B, RBLK), 0)
         ).astype(jnp.float32)
    ssum[...] += jnp.dot(P, z, preferred_element_type=jnp.float32)
    cnt[...] += jnp.broadcast_to(jnp.sum(P, axis=1, keepdims=True), (B, D))

    @pl.when(i == GR - 1)
    def _heads():
        g = ssum[...] / jnp.maximum(cnt[...], 1.0)
        th = jnp.maximum(
            jnp.dot(g, wt1_ref[...], preferred_element_type=jnp.float32)
            + bt1_ref[...], 0.0)
        type_ref[...] = jnp.dot(th, wt2_ref[...],
                                preferred_element_type=jnp.float32) + bt2_ref[...]
        for t in range(T):
            hg = jnp.maximum(
                jnp.dot(g, wg1_ref[t], preferred_element_type=jnp.float32)
                + bg1_ref[t:t + 1, :], 0.0)
            grade_ref[t] = jnp.dot(hg, wg2_ref[t],
                                   preferred_element_type=jnp.float32
                                   ) + bg2_ref[t:t + 1, :]


def _fin(aggp, hp2, dv, b2, bat3, Wt1, bt1, Wt2p, bt2p, Wg1, bg1, Wg2p, bg2p):
    return pl.pallas_call(
        _fin_body,
        grid=(GR,),
        in_specs=[
            pl.BlockSpec((NC, RBLK, D), lambda i: (0, i, 0)),
            pl.BlockSpec((RBLK, D), lambda i: (i, 0)),
            pl.BlockSpec((RBLK, 1), lambda i: (i, 0)),
            pl.BlockSpec((1, D), lambda i: (0, 0)),
            pl.BlockSpec((1, 1, RBLK), lambda i: (i, 0, 0)),
            pl.BlockSpec((D, D), lambda i: (0, 0)),
            pl.BlockSpec((1, D), lambda i: (0, 0)),
            pl.BlockSpec((D, D), lambda i: (0, 0)),
            pl.BlockSpec((1, D), lambda i: (0, 0)),
            pl.BlockSpec((T, D, D), lambda i: (0, 0, 0)),
            pl.BlockSpec((T, D), lambda i: (0, 0)),
            pl.BlockSpec((T, D, D), lambda i: (0, 0, 0)),
            pl.BlockSpec((T, D), lambda i: (0, 0)),
        ],
        out_specs=[
            pl.BlockSpec((B, D), lambda i: (0, 0)),
            pl.BlockSpec((T, B, D), lambda i: (0, 0, 0)),
        ],
        out_shape=[
            jax.ShapeDtypeStruct((B, D), jnp.float32),
            jax.ShapeDtypeStruct((T, B, D), jnp.float32),
        ],
        scratch_shapes=[
            pltpu.VMEM((B, D), jnp.float32),
            pltpu.VMEM((B, D), jnp.float32),
        ],
        compiler_params=pltpu.CompilerParams(
            dimension_semantics=("arbitrary",)),
    )(aggp, hp2, dv, b2, bat3, Wt1, bt1, Wt2p, bt2p, Wg1, bg1, Wg2p, bg2p)


def kernel(x, edge_index, batch, W1, b1, W2, b2, Wt1, bt1, Wt2, bt2,
           Wg1, bg1, Wg2, bg2):
    f32 = jnp.float32
    ei = edge_index.astype(jnp.int32)
    xp = jnp.zeros((NP, D), f32).at[:N].set(x.astype(f32))
    batp = jnp.concatenate(
        [batch.astype(jnp.int32), jnp.full((NP - N,), B, jnp.int32)]
    ).reshape(GR, 1, RBLK)
    ones_rows = jnp.ones((K, D), f32)
    zrows = jnp.zeros((K, D), f32)

    degp = _sc_deg(ei, ones_rows, zrows)
    hp1, dv = _mm1(xp, W1, degp)
    aggp1 = _sc_agg(hp1, ei, zrows)
    hp2 = _mid(aggp1, hp1, dv, b1.reshape(1, D), W2)
    aggp2 = _sc_agg(hp2, ei, zrows)

    Wt2p = jnp.zeros((D, D), f32).at[:, :T].set(Wt2)
    bt2p = jnp.zeros((1, D), f32).at[0, :T].set(bt2)
    Wg2p = jnp.zeros((T, D, D), f32).at[:, :, :G].set(Wg2)
    bg2p = jnp.zeros((T, D), f32).at[:, :G].set(bg2)

    type_full, grade_full = _fin(
        aggp2, hp2, dv, b2.reshape(1, D), batp,
        Wt1, bt1.reshape(1, D), Wt2p, bt2p, Wg1, bg1, Wg2p, bg2p)

    type_logits = type_full[:, :T]
    grade_logits = jnp.transpose(grade_full[:, :, :G], (1, 0, 2))
    return (type_logits, grade_logits)


# speedup vs baseline: 28.2235x; 1.0781x over previous
import functools

import jax
import jax.numpy as jnp
from jax import lax
from jax.experimental import pallas as pl
from jax.experimental.pallas import tpu as pltpu
from jax.experimental.pallas import tpu_sc as plsc

N = 10000
E = 320000
D = 128
B = 64
T = 8
G = 4

NP = 10240
NC = 2
NS = 16
NTILES = NC * NS
K = 128
EPT = 10240
NCH = EPT // K
EP = NTILES * EPT
RPT = NP // NS

RBLK = 1024
GR = NP // RBLK


def _sc_mesh():
    return plsc.VectorSubcoreMesh(
        core_axis_name="c", subcore_axis_name="s",
        num_cores=NC, num_subcores=NS)


def _sc_deg(ei, ones_rows, zrows):
    @functools.partial(
        pl.kernel,
        out_type=jax.ShapeDtypeStruct((NC, NP, D), jnp.float32),
        mesh=_sc_mesh(),
        scratch_types=[
            pltpu.VMEM((K,), jnp.int32),
            pltpu.VMEM((K,), jnp.int32),
            pltpu.VMEM((K, D), jnp.float32),
            pltpu.VMEM((K, D), jnp.float32),
            pltpu.VMEM_SHARED((NP, D), jnp.float32),
            pltpu.SemaphoreType.DMA,
            pltpu.SemaphoreType.DMA,
        ],
    )
    def body(ei_hbm, ones_hbm, zeros_hbm, out_hbm,
             d0, d1, ones_v, stage_v, acc, id0, id1):
        c = lax.axis_index("c")
        s = lax.axis_index("s")
        wid = c * NS + s
        base = wid * EPT
        npair = NCH // 2
        pltpu.sync_copy(ones_hbm, ones_v)
        pltpu.sync_copy(zeros_hbm, stage_v)
        for t in range(RPT // K):
            pltpu.sync_copy(stage_v, acc.at[pl.ds(s * RPT + t * K, K)])
        plsc.subcore_barrier()

        pltpu.async_copy(ei_hbm.at[1, pl.ds(base, K)], d0, id0)
        pltpu.async_copy(ei_hbm.at[1, pl.ds(base + K, K)], d1, id1)

        def step(jj, carry):
            j = jj * 2
            pltpu.make_async_copy(ei_hbm.at[1, pl.ds(base, K)], d0, id0).wait()
            pltpu.sync_copy(ones_v, acc.at[d0], add=True)

            @pl.when(jj < npair - 1)
            def _next0():
                pltpu.async_copy(
                    ei_hbm.at[1, pl.ds(base + (j + 2) * K, K)], d0, id0)

            pltpu.make_async_copy(ei_hbm.at[1, pl.ds(base, K)], d1, id1).wait()
            pltpu.sync_copy(ones_v, acc.at[d1], add=True)

            @pl.when(jj < npair - 1)
            def _next1():
                pltpu.async_copy(
                    ei_hbm.at[1, pl.ds(base + (j + 3) * K, K)], d1, id1)
            return carry
        lax.fori_loop(0, npair, step, 0)
        plsc.subcore_barrier()
        for t in range(RPT // K):
            sl = pl.ds(s * RPT + t * K, K)
            pltpu.sync_copy(acc.at[sl], stage_v)
            pltpu.sync_copy(stage_v, out_hbm.at[c, sl])

    return body(ei, ones_rows, zrows)


def _sc_agg(hp, ei, zrows):
    @functools.partial(
        pl.kernel,
        out_type=jax.ShapeDtypeStruct((NC, NP, D), jnp.float32),
        mesh=_sc_mesh(),
        scratch_types=[
            pltpu.VMEM((EPT,), jnp.int32),
            pltpu.VMEM((K,), jnp.int32),
            pltpu.VMEM((K,), jnp.int32),
            pltpu.VMEM((K, D), jnp.float32),
            pltpu.VMEM((K, D), jnp.float32),
            pltpu.VMEM_SHARED((NP, D), jnp.float32),
            pltpu.SemaphoreType.DMA,
            pltpu.SemaphoreType.DMA,
            pltpu.SemaphoreType.DMA,
            pltpu.SemaphoreType.DMA,
        ],
    )
    def body(hp_hbm, ei_hbm, z_hbm, out_hbm,
             isrc, d0, d1, rows0, rows1, acc, g0, g1, id0, id1):
        c = lax.axis_index("c")
        s = lax.axis_index("s")
        wid = c * NS + s
        base = wid * EPT
        pltpu.sync_copy(ei_hbm.at[0, pl.ds(base, EPT)], isrc)
        pltpu.sync_copy(z_hbm, rows0)
        for t in range(RPT // K):
            pltpu.sync_copy(rows0, acc.at[pl.ds(s * RPT + t * K, K)])
        plsc.subcore_barrier()

        pltpu.async_copy(ei_hbm.at[1, pl.ds(base, K)], d0, id0)
        pltpu.async_copy(ei_hbm.at[1, pl.ds(base + K, K)], d1, id1)
        pltpu.async_copy(hp_hbm.at[isrc.at[pl.ds(0, K)]], rows0, g0)
        pltpu.async_copy(hp_hbm.at[isrc.at[pl.ds(K, K)]], rows1, g1)

        def step(jj, carry):
            j = jj * 2
            more = jj < NCH // 2 - 1
            pltpu.make_async_copy(ei_hbm.at[1, pl.ds(base, K)], d0, id0).wait()
            pltpu.make_async_copy(
                hp_hbm.at[isrc.at[pl.ds(0, K)]], rows0, g0).wait()
            pltpu.sync_copy(rows0, acc.at[d0], add=True)

            @pl.when(more)
            def _next0():
                pltpu.async_copy(
                    ei_hbm.at[1, pl.ds(base + (j + 2) * K, K)], d0, id0)
                pltpu.async_copy(
                    hp_hbm.at[isrc.at[pl.ds((j + 2) * K, K)]], rows0, g0)

            pltpu.make_async_copy(ei_hbm.at[1, pl.ds(base, K)], d1, id1).wait()
            pltpu.make_async_copy(
                hp_hbm.at[isrc.at[pl.ds(0, K)]], rows1, g1).wait()
            pltpu.sync_copy(rows1, acc.at[d1], add=True)

            @pl.when(more)
            def _next1():
                pltpu.async_copy(
                    ei_hbm.at[1, pl.ds(base + (j + 3) * K, K)], d1, id1)
                pltpu.async_copy(
                    hp_hbm.at[isrc.at[pl.ds((j + 3) * K, K)]], rows1, g1)
            return carry
        lax.fori_loop(0, NCH // 2, step, 0)

        plsc.subcore_barrier()
        for t in range(RPT // K):
            sl = pl.ds(s * RPT + t * K, K)
            pltpu.sync_copy(acc.at[sl], rows0)
            pltpu.sync_copy(rows0, out_hbm.at[c, sl])

    return body(hp, ei, zrows)


def _mm1_body(x_ref, w_ref, dp_ref, o_ref, dv_ref):
    deg = dp_ref[0, :, 0:1] + dp_ref[1, :, 0:1] + 1.0
    dv = lax.rsqrt(deg)
    dv_ref[...] = dv
    o_ref[...] = dv * jnp.dot(x_ref[...], w_ref[...],
                              preferred_element_type=jnp.float32)


def _mm1(xp, W1, degp):
    return pl.pallas_call(
        _mm1_body,
        grid=(GR,),
        in_specs=[
            pl.BlockSpec((RBLK, D), lambda i: (i, 0)),
            pl.BlockSpec((D, D), lambda i: (0, 0)),
            pl.BlockSpec((NC, RBLK, D), lambda i: (0, i, 0)),
        ],
        out_specs=[
            pl.BlockSpec((RBLK, D), lambda i: (i, 0)),
            pl.BlockSpec((RBLK, 1), lambda i: (i, 0)),
        ],
        out_shape=[
            jax.ShapeDtypeStruct((NP, D), jnp.float32),
            jax.ShapeDtypeStruct((NP, 1), jnp.float32),
        ],
    )(xp, W1, degp)


def _mid_body(ap_ref, hp_ref, dv_ref, b_ref, w_ref, o_ref):
    dv = dv_ref[...]
    z = jnp.maximum(
        dv * (ap_ref[0] + ap_ref[1] + hp_ref[...]) + b_ref[...], 0.0)
    o_ref[...] = dv * jnp.dot(z, w_ref[...], preferred_element_type=jnp.float32)


def _mid(aggp, hp1, dv, b1, W2):
    return pl.pallas_call(
        _mid_body,
        grid=(GR,),
        in_specs=[
            pl.BlockSpec((NC, RBLK, D), lambda i: (0, i, 0)),
            pl.BlockSpec((RBLK, D), lambda i: (i, 0)),
            pl.BlockSpec((RBLK, 1), lambda i: (i, 0)),
            pl.BlockSpec((1, D), lambda i: (0, 0)),
            pl.BlockSpec((D, D), lambda i: (0, 0)),
        ],
        out_specs=pl.BlockSpec((RBLK, D), lambda i: (i, 0)),
        out_shape=jax.ShapeDtypeStruct((NP, D), jnp.float32),
    )(aggp, hp1, dv, b1, W2)


def _fin_body(ap_ref, hp_ref, dv_ref, b2_ref, bat_ref,
              wt1_ref, bt1_ref, wt2_ref, bt2_ref,
              wg1_ref, bg1_ref, wg2_ref, bg2_ref,
              type_ref, grade_ref, ssum, cnt):
    i = pl.program_id(0)

    @pl.when(i == 0)
    def _init():
        ssum[...] = jnp.zeros_like(ssum)
        cnt[...] = jnp.zeros_like(cnt)

    dv = dv_ref[...]
    z = jnp.maximum(
        dv * (ap_ref[0] + ap_ref[1] + hp_ref[...]) + b2_ref[...], 0.0)
    bvec = bat_ref[0, 0, :]
    P = (bvec[None, :] == lax.broadcasted_iota(jnp.int32, (B, RBLK), 0)
         ).astype(jnp.float32)
    ssum[...] += jnp.dot(P, z, preferred_element_type=jnp.float32)
    cnt[...] += jnp.broadcast_to(jnp.sum(P, axis=1, keepdims=True), (B, D))

    @pl.when(i == GR - 1)
    def _heads():
        g = ssum[...] / jnp.maximum(cnt[...], 1.0)
        th = jnp.maximum(
            jnp.dot(g, wt1_ref[...], preferred_element_type=jnp.float32)
            + bt1_ref[...], 0.0)
        type_ref[...] = jnp.dot(th, wt2_ref[...],
                                preferred_element_type=jnp.float32) + bt2_ref[...]
        for t in range(T):
            hg = jnp.maximum(
                jnp.dot(g, wg1_ref[t], preferred_element_type=jnp.float32)
                + bg1_ref[t:t + 1, :], 0.0)
            grade_ref[t] = jnp.dot(hg, wg2_ref[t],
                                   preferred_element_type=jnp.float32
                                   ) + bg2_ref[t:t + 1, :]


def _fin(aggp, hp2, dv, b2, bat3, Wt1, bt1, Wt2p, bt2p, Wg1, bg1, Wg2p, bg2p):
    return pl.pallas_call(
        _fin_body,
        grid=(GR,),
        in_specs=[
            pl.BlockSpec((NC, RBLK, D), lambda i: (0, i, 0)),
            pl.BlockSpec((RBLK, D), lambda i: (i, 0)),
            pl.BlockSpec((RBLK, 1), lambda i: (i, 0)),
            pl.BlockSpec((1, D), lambda i: (0, 0)),
            pl.BlockSpec((1, 1, RBLK), lambda i: (i, 0, 0)),
            pl.BlockSpec((D, D), lambda i: (0, 0)),
            pl.BlockSpec((1, D), lambda i: (0, 0)),
            pl.BlockSpec((D, D), lambda i: (0, 0)),
            pl.BlockSpec((1, D), lambda i: (0, 0)),
            pl.BlockSpec((T, D, D), lambda i: (0, 0, 0)),
            pl.BlockSpec((T, D), lambda i: (0, 0)),
            pl.BlockSpec((T, D, D), lambda i: (0, 0, 0)),
            pl.BlockSpec((T, D), lambda i: (0, 0)),
        ],
        out_specs=[
            pl.BlockSpec((B, D), lambda i: (0, 0)),
            pl.BlockSpec((T, B, D), lambda i: (0, 0, 0)),
        ],
        out_shape=[
            jax.ShapeDtypeStruct((B, D), jnp.float32),
            jax.ShapeDtypeStruct((T, B, D), jnp.float32),
        ],
        scratch_shapes=[
            pltpu.VMEM((B, D), jnp.float32),
            pltpu.VMEM((B, D), jnp.float32),
        ],
        compiler_params=pltpu.CompilerParams(
            dimension_semantics=("arbitrary",)),
    )(aggp, hp2, dv, b2, bat3, Wt1, bt1, Wt2p, bt2p, Wg1, bg1, Wg2p, bg2p)


def kernel(x, edge_index, batch, W1, b1, W2, b2, Wt1, bt1, Wt2, bt2,
           Wg1, bg1, Wg2, bg2):
    f32 = jnp.float32
    pad_e = EP - E
    pad_idx = N + jnp.arange(pad_e, dtype=jnp.int32) % (NP - N)
    ei = jnp.concatenate(
        [edge_index.astype(jnp.int32),
         jnp.stack([pad_idx, pad_idx])], axis=1)
    xp = jnp.zeros((NP, D), f32).at[:N].set(x.astype(f32))
    batp = jnp.concatenate(
        [batch.astype(jnp.int32), jnp.full((NP - N,), B, jnp.int32)]
    ).reshape(GR, 1, RBLK)
    ones_rows = jnp.ones((K, D), f32)
    zrows = jnp.zeros((K, D), f32)

    degp = _sc_deg(ei, ones_rows, zrows)
    hp1, dv = _mm1(xp, W1, degp)
    aggp1 = _sc_agg(hp1, ei, zrows)
    hp2 = _mid(aggp1, hp1, dv, b1.reshape(1, D), W2)
    aggp2 = _sc_agg(hp2, ei, zrows)

    Wt2p = jnp.zeros((D, D), f32).at[:, :T].set(Wt2)
    bt2p = jnp.zeros((1, D), f32).at[0, :T].set(bt2)
    Wg2p = jnp.zeros((T, D, D), f32).at[:, :, :G].set(Wg2)
    bg2p = jnp.zeros((T, D), f32).at[:, :G].set(bg2)

    type_full, grade_full = _fin(
        aggp2, hp2, dv, b2.reshape(1, D), batp,
        Wt1, bt1.reshape(1, D), Wt2p, bt2p, Wg1, bg1, Wg2p, bg2p)

    type_logits = type_full[:, :T]
    grade_logits = jnp.transpose(grade_full[:, :, :G], (1, 0, 2))
    return (type_logits, grade_logits)


# speedup vs baseline: 29.0932x; 1.0308x over previous
import functools

import jax
import jax.numpy as jnp
from jax import lax
from jax.experimental import pallas as pl
from jax.experimental.pallas import tpu as pltpu
from jax.experimental.pallas import tpu_sc as plsc

N = 10000
E = 320000
D = 128
B = 64
T = 8
G = 4

NP = 10240
NC = 2
NS = 16
NTILES = NC * NS
K = 128
EPT = 10240
NCH = EPT // K
EP = NTILES * EPT
RPT = NP // NS

RBLK = 1024
GR = NP // RBLK


def _sc_mesh():
    return plsc.VectorSubcoreMesh(
        core_axis_name="c", subcore_axis_name="s",
        num_cores=NC, num_subcores=NS)


def _sc_deg(ei, ones_rows, zrows):
    @functools.partial(
        pl.kernel,
        out_type=jax.ShapeDtypeStruct((NC, NP, D), jnp.float32),
        mesh=_sc_mesh(),
        scratch_types=[
            pltpu.VMEM((K,), jnp.int32),
            pltpu.VMEM((K,), jnp.int32),
            pltpu.VMEM((K, D), jnp.float32),
            pltpu.VMEM((K, D), jnp.float32),
            pltpu.VMEM_SHARED((NP, D), jnp.float32),
            pltpu.SemaphoreType.DMA,
            pltpu.SemaphoreType.DMA,
        ],
    )
    def body(ei_hbm, ones_hbm, zeros_hbm, out_hbm,
             d0, d1, ones_v, stage_v, acc, id0, id1):
        c = lax.axis_index("c")
        s = lax.axis_index("s")
        wid = c * NS + s
        base = wid * EPT
        npair = NCH // 2
        pltpu.async_copy(ei_hbm.at[1, pl.ds(base, K)], d0, id0)
        pltpu.async_copy(ei_hbm.at[1, pl.ds(base + K, K)], d1, id1)
        pltpu.sync_copy(ones_hbm, ones_v)
        pltpu.sync_copy(zeros_hbm, stage_v)
        for t in range(RPT // K):
            pltpu.sync_copy(stage_v, acc.at[pl.ds(s * RPT + t * K, K)])
        plsc.subcore_barrier()

        def step(jj, carry):
            j = jj * 2
            pltpu.make_async_copy(ei_hbm.at[1, pl.ds(base, K)], d0, id0).wait()
            pltpu.sync_copy(ones_v, acc.at[d0], add=True)

            @pl.when(jj < npair - 1)
            def _next0():
                pltpu.async_copy(
                    ei_hbm.at[1, pl.ds(base + (j + 2) * K, K)], d0, id0)

            pltpu.make_async_copy(ei_hbm.at[1, pl.ds(base, K)], d1, id1).wait()
            pltpu.sync_copy(ones_v, acc.at[d1], add=True)

            @pl.when(jj < npair - 1)
            def _next1():
                pltpu.async_copy(
                    ei_hbm.at[1, pl.ds(base + (j + 3) * K, K)], d1, id1)
            return carry
        lax.fori_loop(0, npair, step, 0)
        plsc.subcore_barrier()
        stg = (stage_v, ones_v)
        wsem = (id0, id1)
        nt = RPT // K
        for t in range(nt):
            sl = pl.ds(s * RPT + t * K, K)
            if t >= 2:
                slp = pl.ds(s * RPT + (t - 2) * K, K)
                pltpu.make_async_copy(
                    stg[t % 2], out_hbm.at[c, slp], wsem[t % 2]).wait()
            pltpu.sync_copy(acc.at[sl], stg[t % 2])
            pltpu.async_copy(stg[t % 2], out_hbm.at[c, sl], wsem[t % 2])
        for t in (nt - 2, nt - 1):
            sl = pl.ds(s * RPT + t * K, K)
            pltpu.make_async_copy(
                stg[t % 2], out_hbm.at[c, sl], wsem[t % 2]).wait()

    return body(ei, ones_rows, zrows)


def _sc_agg(hp, ei, zrows):
    @functools.partial(
        pl.kernel,
        out_type=jax.ShapeDtypeStruct((NC, NP, D), jnp.float32),
        mesh=_sc_mesh(),
        scratch_types=[
            pltpu.VMEM((EPT,), jnp.int32),
            pltpu.VMEM((K,), jnp.int32),
            pltpu.VMEM((K,), jnp.int32),
            pltpu.VMEM((K, D), jnp.float32),
            pltpu.VMEM((K, D), jnp.float32),
            pltpu.VMEM((32, D), jnp.float32),
            pltpu.VMEM_SHARED((NP, D), jnp.float32),
            pltpu.SemaphoreType.DMA,
            pltpu.SemaphoreType.DMA,
            pltpu.SemaphoreType.DMA,
            pltpu.SemaphoreType.DMA,
            pltpu.SemaphoreType.DMA,
        ],
    )
    def body(hp_hbm, ei_hbm, z_hbm, out_hbm,
             isrc, d0, d1, rows0, rows1, zbuf, acc, g0, g1, id0, id1, zs):
        c = lax.axis_index("c")
        s = lax.axis_index("s")
        wid = c * NS + s
        base = wid * EPT
        pltpu.sync_copy(ei_hbm.at[0, pl.ds(base, EPT)], isrc)
        pltpu.async_copy(ei_hbm.at[1, pl.ds(base, K)], d0, id0)
        pltpu.async_copy(ei_hbm.at[1, pl.ds(base + K, K)], d1, id1)
        pltpu.async_copy(hp_hbm.at[isrc.at[pl.ds(0, K)]], rows0, g0)
        pltpu.async_copy(hp_hbm.at[isrc.at[pl.ds(K, K)]], rows1, g1)
        pltpu.sync_copy(z_hbm.at[pl.ds(0, 32)], zbuf)
        for t in range(RPT // 32):
            pltpu.async_copy(zbuf, acc.at[pl.ds(s * RPT + t * 32, 32)], zs)
        for t in range(RPT // 32):
            pltpu.make_async_copy(zbuf, acc.at[pl.ds(t * 32, 32)], zs).wait()
        plsc.subcore_barrier()

        def step(jj, carry):
            j = jj * 2
            more = jj < NCH // 2 - 1
            pltpu.make_async_copy(ei_hbm.at[1, pl.ds(base, K)], d0, id0).wait()
            pltpu.make_async_copy(
                hp_hbm.at[isrc.at[pl.ds(0, K)]], rows0, g0).wait()
            pltpu.sync_copy(rows0, acc.at[d0], add=True)

            @pl.when(more)
            def _next0():
                pltpu.async_copy(
                    ei_hbm.at[1, pl.ds(base + (j + 2) * K, K)], d0, id0)
                pltpu.async_copy(
                    hp_hbm.at[isrc.at[pl.ds((j + 2) * K, K)]], rows0, g0)

            pltpu.make_async_copy(ei_hbm.at[1, pl.ds(base, K)], d1, id1).wait()
            pltpu.make_async_copy(
                hp_hbm.at[isrc.at[pl.ds(0, K)]], rows1, g1).wait()
            pltpu.sync_copy(rows1, acc.at[d1], add=True)

            @pl.when(more)
            def _next1():
                pltpu.async_copy(
                    ei_hbm.at[1, pl.ds(base + (j + 3) * K, K)], d1, id1)
                pltpu.async_copy(
                    hp_hbm.at[isrc.at[pl.ds((j + 3) * K, K)]], rows1, g1)
            return carry
        lax.fori_loop(0, NCH // 2, step, 0)

        plsc.subcore_barrier()
        rowsb = (rows0, rows1)
        wsem = (g0, g1)
        nt = RPT // K
        for t in range(nt):
            sl = pl.ds(s * RPT + t * K, K)
            if t >= 2:
                slp = pl.ds(s * RPT + (t - 2) * K, K)
                pltpu.make_async_copy(
                    rowsb[t % 2], out_hbm.at[c, slp], wsem[t % 2]).wait()
            pltpu.sync_copy(acc.at[sl], rowsb[t % 2])
            pltpu.async_copy(rowsb[t % 2], out_hbm.at[c, sl], wsem[t % 2])
        for t in (nt - 2, nt - 1):
            sl = pl.ds(s * RPT + t * K, K)
            pltpu.make_async_copy(
                rowsb[t % 2], out_hbm.at[c, sl], wsem[t % 2]).wait()

    return body(hp, ei, zrows)


def _mm1_body(x_ref, w_ref, dp_ref, o_ref, dv_ref):
    deg = dp_ref[0, :, 0:1] + dp_ref[1, :, 0:1] + 1.0
    dv = lax.rsqrt(deg)
    dv_ref[...] = dv
    o_ref[...] = dv * jnp.dot(x_ref[...], w_ref[...],
                              preferred_element_type=jnp.float32)


def _mm1(xp, W1, degp):
    return pl.pallas_call(
        _mm1_body,
        grid=(GR,),
        in_specs=[
            pl.BlockSpec((RBLK, D), lambda i: (i, 0)),
            pl.BlockSpec((D, D), lambda i: (0, 0)),
            pl.BlockSpec((NC, RBLK, D), lambda i: (0, i, 0)),
        ],
        out_specs=[
            pl.BlockSpec((RBLK, D), lambda i: (i, 0)),
            pl.BlockSpec((RBLK, 1), lambda i: (i, 0)),
        ],
        out_shape=[
            jax.ShapeDtypeStruct((NP, D), jnp.float32),
            jax.ShapeDtypeStruct((NP, 1), jnp.float32),
        ],
    )(xp, W1, degp)


def _mid_body(ap_ref, hp_ref, dv_ref, b_ref, w_ref, o_ref):
    dv = dv_ref[...]
    z = jnp.maximum(
        dv * (ap_ref[0] + ap_ref[1] + hp_ref[...]) + b_ref[...], 0.0)
    o_ref[...] = dv * jnp.dot(z, w_ref[...], preferred_element_type=jnp.float32)


def _mid(aggp, hp1, dv, b1, W2):
    return pl.pallas_call(
        _mid_body,
        grid=(GR,),
        in_specs=[
            pl.BlockSpec((NC, RBLK, D), lambda i: (0, i, 0)),
            pl.BlockSpec((RBLK, D), lambda i: (i, 0)),
            pl.BlockSpec((RBLK, 1), lambda i: (i, 0)),
            pl.BlockSpec((1, D), lambda i: (0, 0)),
            pl.BlockSpec((D, D), lambda i: (0, 0)),
        ],
        out_specs=pl.BlockSpec((RBLK, D), lambda i: (i, 0)),
        out_shape=jax.ShapeDtypeStruct((NP, D), jnp.float32),
    )(aggp, hp1, dv, b1, W2)


def _fin_body(ap_ref, hp_ref, dv_ref, b2_ref, bat_ref,
              wt1_ref, bt1_ref, wt2_ref, bt2_ref,
              wg1_ref, bg1_ref, wg2_ref, bg2_ref,
              type_ref, grade_ref, ssum, cnt):
    i = pl.program_id(0)

    @pl.when(i == 0)
    def _init():
        ssum[...] = jnp.zeros_like(ssum)
        cnt[...] = jnp.zeros_like(cnt)

    dv = dv_ref[...]
    z = jnp.maximum(
        dv * (ap_ref[0] + ap_ref[1] + hp_ref[...]) + b2_ref[...], 0.0)
    bvec = bat_ref[0, 0, :]
    P = (bvec[None, :] == lax.broadcasted_iota(jnp.int32, (B, RBLK), 0)
         ).astype(jnp.float32)
    ssum[...] += jnp.dot(P, z, preferred_element_type=jnp.float32)
    cnt[...] += jnp.broadcast_to(jnp.sum(P, axis=1, keepdims=True), (B, D))

    @pl.when(i == GR - 1)
    def _heads():
        g = ssum[...] / jnp.maximum(cnt[...], 1.0)
        th = jnp.maximum(
            jnp.dot(g, wt1_ref[...], preferred_element_type=jnp.float32)
            + bt1_ref[...], 0.0)
        type_ref[...] = jnp.dot(th, wt2_ref[...],
                                preferred_element_type=jnp.float32) + bt2_ref[...]
        for t in range(T):
            hg = jnp.maximum(
                jnp.dot(g, wg1_ref[t], preferred_element_type=jnp.float32)
                + bg1_ref[t:t + 1, :], 0.0)
            grade_ref[t] = jnp.dot(hg, wg2_ref[t],
                                   preferred_element_type=jnp.float32
                                   ) + bg2_ref[t:t + 1, :]


def _fin(aggp, hp2, dv, b2, bat3, Wt1, bt1, Wt2p, bt2p, Wg1, bg1, Wg2p, bg2p):
    return pl.pallas_call(
        _fin_body,
        grid=(GR,),
        in_specs=[
            pl.BlockSpec((NC, RBLK, D), lambda i: (0, i, 0)),
            pl.BlockSpec((RBLK, D), lambda i: (i, 0)),
            pl.BlockSpec((RBLK, 1), lambda i: (i, 0)),
            pl.BlockSpec((1, D), lambda i: (0, 0)),
            pl.BlockSpec((1, 1, RBLK), lambda i: (i, 0, 0)),
            pl.BlockSpec((D, D), lambda i: (0, 0)),
            pl.BlockSpec((1, D), lambda i: (0, 0)),
            pl.BlockSpec((D, D), lambda i: (0, 0)),
            pl.BlockSpec((1, D), lambda i: (0, 0)),
            pl.BlockSpec((T, D, D), lambda i: (0, 0, 0)),
            pl.BlockSpec((T, D), lambda i: (0, 0)),
            pl.BlockSpec((T, D, D), lambda i: (0, 0, 0)),
            pl.BlockSpec((T, D), lambda i: (0, 0)),
        ],
        out_specs=[
            pl.BlockSpec((B, D), lambda i: (0, 0)),
            pl.BlockSpec((T, B, D), lambda i: (0, 0, 0)),
        ],
        out_shape=[
            jax.ShapeDtypeStruct((B, D), jnp.float32),
            jax.ShapeDtypeStruct((T, B, D), jnp.float32),
        ],
        scratch_shapes=[
            pltpu.VMEM((B, D), jnp.float32),
            pltpu.VMEM((B, D), jnp.float32),
        ],
        compiler_params=pltpu.CompilerParams(
            dimension_semantics=("arbitrary",)),
    )(aggp, hp2, dv, b2, bat3, Wt1, bt1, Wt2p, bt2p, Wg1, bg1, Wg2p, bg2p)


def kernel(x, edge_index, batch, W1, b1, W2, b2, Wt1, bt1, Wt2, bt2,
           Wg1, bg1, Wg2, bg2):
    f32 = jnp.float32
    pad_e = EP - E
    pad_idx = N + jnp.arange(pad_e, dtype=jnp.int32) % (NP - N)
    ei = jnp.concatenate(
        [edge_index.astype(jnp.int32),
         jnp.stack([pad_idx, pad_idx])], axis=1)
    xp = jnp.zeros((NP, D), f32).at[:N].set(x.astype(f32))
    batp = jnp.concatenate(
        [batch.astype(jnp.int32), jnp.full((NP - N,), B, jnp.int32)]
    ).reshape(GR, 1, RBLK)
    ones_rows = jnp.ones((K, D), f32)
    zrows = jnp.zeros((K, D), f32)

    degp = _sc_deg(ei, ones_rows, zrows)
    hp1, dv = _mm1(xp, W1, degp)
    aggp1 = _sc_agg(hp1, ei, zrows)
    hp2 = _mid(aggp1, hp1, dv, b1.reshape(1, D), W2)
    aggp2 = _sc_agg(hp2, ei, zrows)

    Wt2p = jnp.zeros((D, D), f32).at[:, :T].set(Wt2)
    bt2p = jnp.zeros((1, D), f32).at[0, :T].set(bt2)
    Wg2p = jnp.zeros((T, D, D), f32).at[:, :, :G].set(Wg2)
    bg2p = jnp.zeros((T, D), f32).at[:, :G].set(bg2)

    type_full, grade_full = _fin(
        aggp2, hp2, dv, b2.reshape(1, D), batp,
        Wt1, bt1.reshape(1, D), Wt2p, bt2p, Wg1, bg1, Wg2p, bg2p)

    type_logits = type_full[:, :T]
    grade_logits = jnp.transpose(grade_full[:, :, :G], (1, 0, 2))
    return (type_logits, grade_logits)


# speedup vs baseline: 29.1827x; 1.0031x over previous
import functools

import jax
import jax.numpy as jnp
from jax import lax
from jax.experimental import pallas as pl
from jax.experimental.pallas import tpu as pltpu
from jax.experimental.pallas import tpu_sc as plsc

N = 10000
E = 320000
D = 128
B = 64
T = 8
G = 4

NP = 10240
NC = 2
NS = 16
NTILES = NC * NS
K = 128
EPT = 10240
NCH = EPT // K
EP = NTILES * EPT
RPT = NP // NS

RBLK = 1000
GR = N // RBLK


def _sc_mesh():
    return plsc.VectorSubcoreMesh(
        core_axis_name="c", subcore_axis_name="s",
        num_cores=NC, num_subcores=NS)


def _sc_deg(ei, ones_rows, zrows):
    @functools.partial(
        pl.kernel,
        out_type=jax.ShapeDtypeStruct((NC, NP, D), jnp.float32),
        mesh=_sc_mesh(),
        scratch_types=[
            pltpu.VMEM((K,), jnp.int32),
            pltpu.VMEM((K,), jnp.int32),
            pltpu.VMEM((K, D), jnp.float32),
            pltpu.VMEM((K, D), jnp.float32),
            pltpu.VMEM_SHARED((NP, D), jnp.float32),
            pltpu.SemaphoreType.DMA,
            pltpu.SemaphoreType.DMA,
        ],
    )
    def body(ei_hbm, ones_hbm, zeros_hbm, out_hbm,
             d0, d1, ones_v, stage_v, acc, id0, id1):
        c = lax.axis_index("c")
        s = lax.axis_index("s")
        wid = c * NS + s
        base = wid * EPT
        npair = NCH // 2
        pltpu.async_copy(ei_hbm.at[1, pl.ds(base, K)], d0, id0)
        pltpu.async_copy(ei_hbm.at[1, pl.ds(base + K, K)], d1, id1)
        pltpu.sync_copy(ones_hbm, ones_v)
        pltpu.sync_copy(zeros_hbm, stage_v)
        for t in range(RPT // K):
            pltpu.sync_copy(stage_v, acc.at[pl.ds(s * RPT + t * K, K)])
        plsc.subcore_barrier()

        def step(jj, carry):
            j = jj * 2
            pltpu.make_async_copy(ei_hbm.at[1, pl.ds(base, K)], d0, id0).wait()
            pltpu.sync_copy(ones_v, acc.at[d0], add=True)

            @pl.when(jj < npair - 1)
            def _next0():
                pltpu.async_copy(
                    ei_hbm.at[1, pl.ds(base + (j + 2) * K, K)], d0, id0)

            pltpu.make_async_copy(ei_hbm.at[1, pl.ds(base, K)], d1, id1).wait()
            pltpu.sync_copy(ones_v, acc.at[d1], add=True)

            @pl.when(jj < npair - 1)
            def _next1():
                pltpu.async_copy(
                    ei_hbm.at[1, pl.ds(base + (j + 3) * K, K)], d1, id1)
            return carry
        lax.fori_loop(0, npair, step, 0)
        plsc.subcore_barrier()
        stg = (stage_v, ones_v)
        wsem = (id0, id1)
        nt = RPT // K
        for t in range(nt):
            sl = pl.ds(s * RPT + t * K, K)
            if t >= 2:
                slp = pl.ds(s * RPT + (t - 2) * K, K)
                pltpu.make_async_copy(
                    stg[t % 2], out_hbm.at[c, slp], wsem[t % 2]).wait()
            pltpu.sync_copy(acc.at[sl], stg[t % 2])
            pltpu.async_copy(stg[t % 2], out_hbm.at[c, sl], wsem[t % 2])
        for t in (nt - 2, nt - 1):
            sl = pl.ds(s * RPT + t * K, K)
            pltpu.make_async_copy(
                stg[t % 2], out_hbm.at[c, sl], wsem[t % 2]).wait()

    return body(ei, ones_rows, zrows)


def _sc_agg(hp, ei, zrows):
    @functools.partial(
        pl.kernel,
        out_type=jax.ShapeDtypeStruct((NC, NP, D), jnp.float32),
        mesh=_sc_mesh(),
        scratch_types=[
            pltpu.VMEM((EPT,), jnp.int32),
            pltpu.VMEM((K,), jnp.int32),
            pltpu.VMEM((K,), jnp.int32),
            pltpu.VMEM((K, D), jnp.float32),
            pltpu.VMEM((K, D), jnp.float32),
            pltpu.VMEM((32, D), jnp.float32),
            pltpu.VMEM_SHARED((NP, D), jnp.float32),
            pltpu.SemaphoreType.DMA,
            pltpu.SemaphoreType.DMA,
            pltpu.SemaphoreType.DMA,
            pltpu.SemaphoreType.DMA,
            pltpu.SemaphoreType.DMA,
        ],
    )
    def body(hp_hbm, ei_hbm, z_hbm, out_hbm,
             isrc, d0, d1, rows0, rows1, zbuf, acc, g0, g1, id0, id1, zs):
        c = lax.axis_index("c")
        s = lax.axis_index("s")
        wid = c * NS + s
        base = wid * EPT
        pltpu.sync_copy(ei_hbm.at[0, pl.ds(base, EPT)], isrc)
        pltpu.async_copy(ei_hbm.at[1, pl.ds(base, K)], d0, id0)
        pltpu.async_copy(ei_hbm.at[1, pl.ds(base + K, K)], d1, id1)
        pltpu.async_copy(hp_hbm.at[isrc.at[pl.ds(0, K)]], rows0, g0)
        pltpu.async_copy(hp_hbm.at[isrc.at[pl.ds(K, K)]], rows1, g1)
        pltpu.sync_copy(z_hbm.at[pl.ds(0, 32)], zbuf)
        for t in range(RPT // 32):
            pltpu.async_copy(zbuf, acc.at[pl.ds(s * RPT + t * 32, 32)], zs)
        for t in range(RPT // 32):
            pltpu.make_async_copy(zbuf, acc.at[pl.ds(t * 32, 32)], zs).wait()
        plsc.subcore_barrier()

        def step(jj, carry):
            j = jj * 2
            more = jj < NCH // 2 - 1
            pltpu.make_async_copy(ei_hbm.at[1, pl.ds(base, K)], d0, id0).wait()
            pltpu.make_async_copy(
                hp_hbm.at[isrc.at[pl.ds(0, K)]], rows0, g0).wait()
            pltpu.sync_copy(rows0, acc.at[d0], add=True)

            @pl.when(more)
            def _next0():
                pltpu.async_copy(
                    ei_hbm.at[1, pl.ds(base + (j + 2) * K, K)], d0, id0)
                pltpu.async_copy(
                    hp_hbm.at[isrc.at[pl.ds((j + 2) * K, K)]], rows0, g0)

            pltpu.make_async_copy(ei_hbm.at[1, pl.ds(base, K)], d1, id1).wait()
            pltpu.make_async_copy(
                hp_hbm.at[isrc.at[pl.ds(0, K)]], rows1, g1).wait()
            pltpu.sync_copy(rows1, acc.at[d1], add=True)

            @pl.when(more)
            def _next1():
                pltpu.async_copy(
                    ei_hbm.at[1, pl.ds(base + (j + 3) * K, K)], d1, id1)
                pltpu.async_copy(
                    hp_hbm.at[isrc.at[pl.ds((j + 3) * K, K)]], rows1, g1)
            return carry
        lax.fori_loop(0, NCH // 2, step, 0)

        plsc.subcore_barrier()
        rowsb = (rows0, rows1)
        wsem = (g0, g1)
        nt = RPT // K
        for t in range(nt):
            sl = pl.ds(s * RPT + t * K, K)
            if t >= 2:
                slp = pl.ds(s * RPT + (t - 2) * K, K)
                pltpu.make_async_copy(
                    rowsb[t % 2], out_hbm.at[c, slp], wsem[t % 2]).wait()
            pltpu.sync_copy(acc.at[sl], rowsb[t % 2])
            pltpu.async_copy(rowsb[t % 2], out_hbm.at[c, sl], wsem[t % 2])
        for t in (nt - 2, nt - 1):
            sl = pl.ds(s * RPT + t * K, K)
            pltpu.make_async_copy(
                rowsb[t % 2], out_hbm.at[c, sl], wsem[t % 2]).wait()

    return body(hp, ei, zrows)


def _mm1_body(x_ref, w_ref, dp_ref, o_ref, dv_ref):
    deg = dp_ref[0, :, 0:1] + dp_ref[1, :, 0:1] + 1.0
    dv = lax.rsqrt(deg)
    dv_ref[...] = dv
    o_ref[...] = dv * jnp.dot(x_ref[...], w_ref[...],
                              preferred_element_type=jnp.float32)


def _mm1(xp, W1, degp):
    return pl.pallas_call(
        _mm1_body,
        grid=(GR,),
        in_specs=[
            pl.BlockSpec((RBLK, D), lambda i: (i, 0)),
            pl.BlockSpec((D, D), lambda i: (0, 0)),
            pl.BlockSpec((NC, RBLK, D), lambda i: (0, i, 0)),
        ],
        out_specs=[
            pl.BlockSpec((RBLK, D), lambda i: (i, 0)),
            pl.BlockSpec((RBLK, 1), lambda i: (i, 0)),
        ],
        out_shape=[
            jax.ShapeDtypeStruct((N, D), jnp.float32),
            jax.ShapeDtypeStruct((N, 1), jnp.float32),
        ],
    )(xp, W1, degp)


def _mid_body(ap_ref, hp_ref, dv_ref, b_ref, w_ref, o_ref):
    dv = dv_ref[...]
    z = jnp.maximum(
        dv * (ap_ref[0] + ap_ref[1] + hp_ref[...]) + b_ref[...], 0.0)
    o_ref[...] = dv * jnp.dot(z, w_ref[...], preferred_element_type=jnp.float32)


def _mid(aggp, hp1, dv, b1, W2):
    return pl.pallas_call(
        _mid_body,
        grid=(GR,),
        in_specs=[
            pl.BlockSpec((NC, RBLK, D), lambda i: (0, i, 0)),
            pl.BlockSpec((RBLK, D), lambda i: (i, 0)),
            pl.BlockSpec((RBLK, 1), lambda i: (i, 0)),
            pl.BlockSpec((1, D), lambda i: (0, 0)),
            pl.BlockSpec((D, D), lambda i: (0, 0)),
        ],
        out_specs=pl.BlockSpec((RBLK, D), lambda i: (i, 0)),
        out_shape=jax.ShapeDtypeStruct((N, D), jnp.float32),
    )(aggp, hp1, dv, b1, W2)


def _fin_body(ap_ref, hp_ref, dv_ref, b2_ref, bat_ref,
              wt1_ref, bt1_ref, wt2_ref, bt2_ref,
              wg1_ref, bg1_ref, wg2_ref, bg2_ref,
              type_ref, grade_ref, ssum, cnt):
    i = pl.program_id(0)

    @pl.when(i == 0)
    def _init():
        ssum[...] = jnp.zeros_like(ssum)
        cnt[...] = jnp.zeros_like(cnt)

    dv = dv_ref[...]
    z = jnp.maximum(
        dv * (ap_ref[0] + ap_ref[1] + hp_ref[...]) + b2_ref[...], 0.0)
    bvec = bat_ref[0, 0, :]
    P = (bvec[None, :] == lax.broadcasted_iota(jnp.int32, (B, RBLK), 0)
         ).astype(jnp.float32)
    ssum[...] += jnp.dot(P, z, preferred_element_type=jnp.float32)
    cnt[...] += jnp.broadcast_to(jnp.sum(P, axis=1, keepdims=True), (B, D))

    @pl.when(i == GR - 1)
    def _heads():
        g = ssum[...] / jnp.maximum(cnt[...], 1.0)
        th = jnp.maximum(
            jnp.dot(g, wt1_ref[...], preferred_element_type=jnp.float32)
            + bt1_ref[...], 0.0)
        type_ref[...] = jnp.dot(th, wt2_ref[...],
                                preferred_element_type=jnp.float32) + bt2_ref[...]
        for t in range(T):
            hg = jnp.maximum(
                jnp.dot(g, wg1_ref[t], preferred_element_type=jnp.float32)
                + bg1_ref[t:t + 1, :], 0.0)
            grade_ref[t] = jnp.dot(hg, wg2_ref[t],
                                   preferred_element_type=jnp.float32
                                   ) + bg2_ref[t:t + 1, :]


def _fin(aggp, hp2, dv, b2, bat3, Wt1, bt1, Wt2p, bt2p, Wg1, bg1, Wg2p, bg2p):
    return pl.pallas_call(
        _fin_body,
        grid=(GR,),
        in_specs=[
            pl.BlockSpec((NC, RBLK, D), lambda i: (0, i, 0)),
            pl.BlockSpec((RBLK, D), lambda i: (i, 0)),
            pl.BlockSpec((RBLK, 1), lambda i: (i, 0)),
            pl.BlockSpec((1, D), lambda i: (0, 0)),
            pl.BlockSpec((1, 1, RBLK), lambda i: (i, 0, 0)),
            pl.BlockSpec((D, D), lambda i: (0, 0)),
            pl.BlockSpec((1, D), lambda i: (0, 0)),
            pl.BlockSpec((D, D), lambda i: (0, 0)),
            pl.BlockSpec((1, D), lambda i: (0, 0)),
            pl.BlockSpec((T, D, D), lambda i: (0, 0, 0)),
            pl.BlockSpec((T, D), lambda i: (0, 0)),
            pl.BlockSpec((T, D, D), lambda i: (0, 0, 0)),
            pl.BlockSpec((T, D), lambda i: (0, 0)),
        ],
        out_specs=[
            pl.BlockSpec((B, D), lambda i: (0, 0)),
            pl.BlockSpec((T, B, D), lambda i: (0, 0, 0)),
        ],
        out_shape=[
            jax.ShapeDtypeStruct((B, D), jnp.float32),
            jax.ShapeDtypeStruct((T, B, D), jnp.float32),
        ],
        scratch_shapes=[
            pltpu.VMEM((B, D), jnp.float32),
            pltpu.VMEM((B, D), jnp.float32),
        ],
        compiler_params=pltpu.CompilerParams(
            dimension_semantics=("arbitrary",)),
    )(aggp, hp2, dv, b2, bat3, Wt1, bt1, Wt2p, bt2p, Wg1, bg1, Wg2p, bg2p)


def kernel(x, edge_index, batch, W1, b1, W2, b2, Wt1, bt1, Wt2, bt2,
           Wg1, bg1, Wg2, bg2):
    f32 = jnp.float32
    pad_e = EP - E
    ar = jnp.arange(pad_e, dtype=jnp.int32)
    ei = jnp.concatenate(
        [edge_index.astype(jnp.int32),
         jnp.stack([ar % N, N + ar % (NP - N)])], axis=1)
    xp = x.astype(f32)
    batp = batch.astype(jnp.int32).reshape(GR, 1, RBLK)
    ones_rows = jnp.ones((K, D), f32)
    zrows = jnp.zeros((K, D), f32)

    degp = _sc_deg(ei, ones_rows, zrows)
    hp1, dv = _mm1(xp, W1, degp)
    aggp1 = _sc_agg(hp1, ei, zrows)
    hp2 = _mid(aggp1, hp1, dv, b1.reshape(1, D), W2)
    aggp2 = _sc_agg(hp2, ei, zrows)

    Wt2p = jnp.zeros((D, D), f32).at[:, :T].set(Wt2)
    bt2p = jnp.zeros((1, D), f32).at[0, :T].set(bt2)
    Wg2p = jnp.zeros((T, D, D), f32).at[:, :, :G].set(Wg2)
    bg2p = jnp.zeros((T, D), f32).at[:, :G].set(bg2)

    type_full, grade_full = _fin(
        aggp2, hp2, dv, b2.reshape(1, D), batp,
        Wt1, bt1.reshape(1, D), Wt2p, bt2p, Wg1, bg1, Wg2p, bg2p)

    type_logits = type_full[:, :T]
    grade_logits = jnp.transpose(grade_full[:, :, :G], (1, 0, 2))
    return (type_logits, grade_logits)
